# baseline (speedup 1.0000x reference)
import functools

import jax
import jax.numpy as jnp
import numpy as np
from jax import lax
from jax.experimental import pallas as pl
from jax.experimental.pallas import tpu as pltpu
from jax.experimental.pallas import tpu_sc as plsc

N = 10000
E = 320000
HID = 128
NH = 16
HD = 8
NG = 20
EF = 4
RF = 80

_OFFSET = np.array([0, 1, 1.25, 1.5, 1.75, 2, 2.25, 2.5, 2.75, 3, 3.5, 4,
                    4.5, 5, 5.5, 6, 7, 8, 9, 10], dtype=np.float32)
_COEFF = -0.5
_INV_SQRT_HD = 1.0 / np.sqrt(HD)

EB = 2000
NEB = E // EB
NB = 2000
NNB = N // NB

NW = 32
EPW = E // NW
GC = 80
NCH = EPW // GC
NP = 10240
ZR = NP // 16

PAY = 128
XP = 128
GW = 256


def _ln(xv, g, b):
    mu = jnp.mean(xv, axis=-1, keepdims=True)
    c = xv - mu
    v = jnp.mean(c * c, axis=-1, keepdims=True)
    return c * lax.rsqrt(v + 1e-5) * g + b


def _seg_matrix(dtype=jnp.float32):
    r = lax.broadcasted_iota(jnp.int32, (HID, NH), 0)
    c = lax.broadcasted_iota(jnp.int32, (HID, NH), 1)
    return (r // HD == c).astype(dtype)


def _expand_matrix(dtype=jnp.float32):
    r = lax.broadcasted_iota(jnp.int32, (NH, HID), 0)
    c = lax.broadcasted_iota(jnp.int32, (NH, HID), 1)
    return (c // HD == r).astype(dtype)


def _node_mlp_body(h_ref, w1t, b1, g1, be1, w2t, b2, o_ref):
    x = h_ref[...]
    h1 = jnp.dot(x, w1t[...], preferred_element_type=jnp.float32) + b1[...]
    h1 = jnp.maximum(_ln(h1, g1[...], be1[...]), 0.0)
    o_ref[...] = jnp.dot(h1, w2t[...], preferred_element_type=jnp.float32) + b2[...]


def _node_mlp(p, h):
    full = lambda s: pl.BlockSpec(s, lambda i: (0, 0))
    return pl.pallas_call(
        _node_mlp_body,
        grid=(NNB,),
        in_specs=[
            pl.BlockSpec((NB, HID), lambda i: (i, 0)),
            full((HID, HID)), full((1, HID)), full((1, HID)), full((1, HID)),
            full((HID, HID)), full((1, HID)),
        ],
        out_specs=pl.BlockSpec((NB, HID), lambda i: (i, 0)),
        out_shape=jax.ShapeDtypeStruct((N, HID), jnp.float32),
    )(h, p['W1'].T, p['b1'].reshape(1, -1), p['g1'].reshape(1, -1),
      p['be1'].reshape(1, -1), p['W2'].T, p['b2'].reshape(1, -1))


def _make_gather(widths, dtype=jnp.float32):
    nt = len(widths)

    def body(*refs):
        idx_hbm = refs[:nt]
        tab_hbm = refs[nt:2 * nt]
        out_hbm = refs[2 * nt:3 * nt]
        scratch = refs[3 * nt:]
        idx_v = scratch[:nt]
        row_v = scratch[nt:2 * nt]
        sem = scratch[2 * nt]
        cid = lax.axis_index("c")
        sid = lax.axis_index("s")
        wid = sid * 2 + cid

        def chunk(i, carry):
            base = wid * EPW + i * GC
            for t in range(nt):
                pltpu.sync_copy(idx_hbm[t].at[pl.ds(base, GC)], idx_v[t].at[0])
            copies = [
                pltpu.async_copy(tab_hbm[t].at[idx_v[t].at[0]], row_v[t], sem)
                for t in range(nt)
            ]
            for c in copies:
                c.wait()
            for t in range(nt):
                pltpu.sync_copy(row_v[t], out_hbm[t].at[pl.ds(base, GC)])
            return carry

        lax.fori_loop(0, NCH, chunk, 0)

    scratch = ([pltpu.VMEM((1, GC), jnp.int32) for _ in range(nt)]
               + [pltpu.VMEM((GC, w), dtype) for w in widths]
               + [pltpu.SemaphoreType.DMA])
    out_type = [jax.ShapeDtypeStruct((E, w), dtype) for w in widths]

    built = {}

    def call(*args):
        if 'k' not in built:
            mesh = plsc.VectorSubcoreMesh(core_axis_name="c",
                                          subcore_axis_name="s")
            built['k'] = functools.partial(
                pl.kernel, mesh=mesh, out_type=out_type,
                scratch_types=scratch)(body)
        return built['k'](*args)

    return call


_gather2i = _make_gather([GW, GW], jnp.int32)


def _pack2bf(a, b):
    au = lax.bitcast_convert_type(a.astype(jnp.bfloat16), jnp.uint16)
    bu = lax.bitcast_convert_type(b.astype(jnp.bfloat16), jnp.uint16)
    w = (au.astype(jnp.uint32) << 16) | bu.astype(jnp.uint32)
    return lax.bitcast_convert_type(w, jnp.int32)


def _unpack_hi(u):
    return lax.bitcast_convert_type(u & jnp.int32(-65536), jnp.float32)


def _unpack_lo(u):
    return lax.bitcast_convert_type(u << 16, jnp.float32)


def _make_scatter(width):
    def body(pay_hbm, didx_hbm, zeros_hbm, out0, out1, idx_v, pay_v, acc):
        cid = lax.axis_index("c")
        sid = lax.axis_index("s")
        wid = sid * 2 + cid
        pltpu.sync_copy(zeros_hbm.at[pl.ds(sid * ZR, ZR)],
                        acc.at[pl.ds(sid * ZR, ZR)])
        plsc.subcore_barrier()

        def chunk(i, carry):
            base = wid * EPW + i * GC
            pltpu.sync_copy(didx_hbm.at[pl.ds(base, GC)], idx_v.at[0])
            pltpu.sync_copy(pay_hbm.at[pl.ds(base, GC)], pay_v)
            pltpu.sync_copy(pay_v, acc.at[idx_v.at[0]], add=True)
            return carry

        lax.fori_loop(0, NCH, chunk, 0)
        plsc.subcore_barrier()

        @pl.when(cid == 0)
        def _():
            pltpu.sync_copy(acc.at[pl.ds(sid * ZR, ZR)],
                            out0.at[pl.ds(sid * ZR, ZR)])

        @pl.when(cid == 1)
        def _():
            pltpu.sync_copy(acc.at[pl.ds(sid * ZR, ZR)],
                            out1.at[pl.ds(sid * ZR, ZR)])

    out_type = [jax.ShapeDtypeStruct((NP, width), jnp.float32)] * 2
    scratch = [pltpu.VMEM((1, GC), jnp.int32),
               pltpu.VMEM((GC, width), jnp.float32),
               pltpu.VMEM_SHARED((NP, width), jnp.float32)]

    built = {}

    def call(*args):
        if 'k' not in built:
            mesh = plsc.VectorSubcoreMesh(core_axis_name="c",
                                          subcore_axis_name="s")
            built['k'] = functools.partial(
                pl.kernel, mesh=mesh, out_type=out_type,
                scratch_types=scratch)(body)
        return built['k'](*args)

    return call


_scatter128 = _make_scatter(PAY)


def _edge_body_a(off_ref, ef_ref, ps_ref, pd_ref,
                 kw1ed, kw1i, kw1j, kb1, kg1, kbe1, kw2t, kb2,
                 vw1ed, vw1i, vw1j, vb1, vg1, vbe1, vw2t, vb2,
                 eww, ewb,
                 v_out, l_out, gm_out):
    ps = ps_ref[...]
    pd = pd_ref[...]
    dx = (lax.bitcast_convert_type(pd[:, HID:], jnp.float32)
          - lax.bitcast_convert_type(ps[:, HID:], jnp.float32))
    dist = jnp.sqrt(jnp.sum(dx * dx, axis=1, keepdims=True))
    off = off_ref[...]
    g = jnp.exp(_COEFF * (dist - off) ** 2)
    ef = ef_ref[...]
    df = jnp.concatenate([ef[:, i:i + 1] * g for i in range(EF)], axis=1)
    edf = jnp.concatenate([ef, df], axis=1).astype(jnp.bfloat16)
    hs = _unpack_hi(ps[:, :HID]).astype(jnp.bfloat16)
    pd = pd[:, :HID]
    hd = _unpack_hi(pd).astype(jnp.bfloat16)

    def mlp(w1ed, w1i, w1j, b1, g1, be1, w2t, b2):
        pre = (jnp.dot(edf, w1ed[...], preferred_element_type=jnp.float32)
               + jnp.dot(hd, w1i[...], preferred_element_type=jnp.float32)
               + jnp.dot(hs, w1j[...], preferred_element_type=jnp.float32)
               + b1[...])
        h1 = jnp.maximum(_ln(pre, g1[...], be1[...]), 0.0).astype(jnp.bfloat16)
        return jnp.dot(h1, w2t[...], preferred_element_type=jnp.float32) + b2[...]

    k = mlp(kw1ed, kw1i, kw1j, kb1, kg1, kbe1, kw2t, kb2)
    v = mlp(vw1ed, vw1i, vw1j, vb1, vg1, vbe1, vw2t, vb2)
    e_w = jax.nn.sigmoid(jnp.sum(df * eww[...], axis=1, keepdims=True)
                         + ewb[...])
    v_out[...] = v * e_w
    qd = _unpack_lo(pd)
    logits = jnp.dot((qd * k) * _INV_SQRT_HD, _seg_matrix(),
                     preferred_element_type=jnp.float32)
    l_out[...] = logits
    bm = jnp.max(logits, axis=0, keepdims=True)

    @pl.when(pl.program_id(0) == 0)
    def _():
        gm_out[...] = jnp.full((8, NH), -1e30, jnp.float32)

    gm_out[...] = jnp.maximum(gm_out[...], bm)


def _edge_a(ef, ps, pd, pk, pv, eww, ewb):
    eb = lambda w: pl.BlockSpec((EB, w), lambda i: (i, 0))
    full = lambda s: pl.BlockSpec(s, lambda i: (0,) * len(s))

    def wspecs():
        return [full((84, HID)), full((HID, HID)), full((HID, HID)),
                full((1, HID)), full((1, HID)), full((1, HID)),
                full((HID, HID)), full((1, HID))]

    def wargs(p):
        w1t = p['W1'].T.astype(jnp.bfloat16)
        return [w1t[0:84], w1t[84:212], w1t[212:340],
                p['b1'].reshape(1, -1), p['g1'].reshape(1, -1),
                p['be1'].reshape(1, -1), p['W2'].T.astype(jnp.bfloat16),
                p['b2'].reshape(1, -1)]

    return pl.pallas_call(
        _edge_body_a,
        grid=(NEB,),
        in_specs=[full((1, NG)), eb(EF), eb(GW), eb(GW)]
                 + wspecs() + wspecs() + [full((1, RF)), full((1, 1))],
        out_specs=[eb(HID), eb(NH), full((8, NH))],
        out_shape=[jax.ShapeDtypeStruct((E, HID), jnp.float32),
                   jax.ShapeDtypeStruct((E, NH), jnp.float32),
                   jax.ShapeDtypeStruct((8, NH), jnp.float32)],
    )(jnp.asarray(_OFFSET).reshape(1, NG), ef, ps, pd,
      *wargs(pk), *wargs(pv), eww.reshape(1, RF), ewb.reshape(1, 1))


def _payload_body_a(l_ref, v_ref, gm_ref, n_ref, d_ref):
    gm = jnp.max(gm_ref[...], axis=0, keepdims=True)
    ex = jnp.exp(l_ref[...] - gm)
    exb = jnp.dot(ex, _expand_matrix(), preferred_element_type=jnp.float32)
    n_ref[...] = v_ref[...] * exb
    d_ref[...] = exb


def _payload_a(logits, v_ew, gmax):
    return pl.pallas_call(
        _payload_body_a,
        grid=(NEB,),
        in_specs=[pl.BlockSpec((EB, NH), lambda i: (i, 0)),
                  pl.BlockSpec((EB, HID), lambda i: (i, 0)),
                  pl.BlockSpec((8, NH), lambda i: (0, 0))],
        out_specs=[pl.BlockSpec((EB, PAY), lambda i: (i, 0)),
                   pl.BlockSpec((EB, PAY), lambda i: (i, 0))],
        out_shape=[jax.ShapeDtypeStruct((E, PAY), jnp.float32),
                   jax.ShapeDtypeStruct((E, PAY), jnp.float32)],
    )(logits, v_ew, gmax)


def _combine_body_a(n0_ref, n1_ref, d0_ref, d1_ref, h_ref,
                    w1a, w1b, b1, g1, be1, w2t, b2, o_ref):
    num = n0_ref[...] + n1_ref[...]
    dnb = d0_ref[...] + d1_ref[...]
    attn = num / jnp.where(dnb == 0.0, 1.0, dnb)
    h = h_ref[...]
    pre = (jnp.dot(attn, w1a[...], preferred_element_type=jnp.float32)
           + jnp.dot(h, w1b[...], preferred_element_type=jnp.float32)
           + b1[...])
    h1 = jnp.maximum(_ln(pre, g1[...], be1[...]), 0.0)
    o_ref[...] = (jnp.dot(h1, w2t[...], preferred_element_type=jnp.float32)
                  + b2[...] + h)


def _combine_a(n0, n1, d0, d1, h, p):
    full = lambda s: pl.BlockSpec(s, lambda i: (0, 0))
    nb = lambda w: pl.BlockSpec((NB, w), lambda i: (i, 0))
    w1t = p['W1'].T
    return pl.pallas_call(
        _combine_body_a,
        grid=(NNB,),
        in_specs=[nb(PAY), nb(PAY), nb(PAY), nb(PAY), nb(HID),
                  full((HID, HID)), full((HID, HID)), full((1, HID)),
                  full((1, HID)), full((1, HID)), full((HID, HID)),
                  full((1, HID))],
        out_specs=pl.BlockSpec((NB, HID), lambda i: (i, 0)),
        out_shape=jax.ShapeDtypeStruct((N, HID), jnp.float32),
    )(n0, n1, d0, d1, h, w1t[:HID], w1t[HID:], p['b1'].reshape(1, -1),
      p['g1'].reshape(1, -1), p['be1'].reshape(1, -1), p['W2'].T,
      p['b2'].reshape(1, -1))


def _edge_body_b(off_ref, ef_ref, ps_ref, pd_ref,
                 kw1ed, kw1i, kw1j, kb1, kg1, kbe1, kw2t, kb2,
                 vw1ed, vw1i, vw1j, vb1, vg1, vbe1, vw2t, vb2,
                 eww, ewb,
                 w_out, l_out, gm_out):
    ps = ps_ref[...]
    pd = pd_ref[...]
    dx = (lax.bitcast_convert_type(pd[:, HID:], jnp.float32)
          - lax.bitcast_convert_type(ps[:, HID:], jnp.float32))
    dist = jnp.sqrt(jnp.sum(dx * dx, axis=1, keepdims=True))
    off = off_ref[...]
    g = jnp.exp(_COEFF * (dist - off) ** 2)
    ef = ef_ref[...]
    df = jnp.concatenate([ef[:, i:i + 1] * g for i in range(EF)], axis=1)
    edf = jnp.concatenate([ef, df], axis=1).astype(jnp.bfloat16)
    hs = _unpack_hi(ps[:, :HID]).astype(jnp.bfloat16)
    pd = pd[:, :HID]
    hd = _unpack_hi(pd).astype(jnp.bfloat16)

    def mlp(w1ed, w1i, w1j, b1, g1, be1, w2t, b2):
        pre = (jnp.dot(edf, w1ed[...], preferred_element_type=jnp.float32)
               + jnp.dot(hd, w1i[...], preferred_element_type=jnp.float32)
               + jnp.dot(hs, w1j[...], preferred_element_type=jnp.float32)
               + b1[...])
        h1 = jnp.maximum(_ln(pre, g1[...], be1[...]), 0.0).astype(jnp.bfloat16)
        return jnp.dot(h1, w2t[...], preferred_element_type=jnp.float32) + b2[...]

    k = mlp(kw1ed, kw1i, kw1j, kb1, kg1, kbe1, kw2t, kb2)
    v2 = mlp(vw1ed, vw1i, vw1j, vb1, vg1, vbe1, vw2t, vb2)
    e_w = jax.nn.sigmoid(jnp.sum(df * eww[...], axis=1, keepdims=True)
                         + ewb[...])
    w_out[...] = v2 * e_w
    qd = _unpack_lo(pd)
    logits = jnp.dot((qd * k) * _INV_SQRT_HD, _seg_matrix(),
                     preferred_element_type=jnp.float32)
    l_out[...] = logits
    bm = jnp.max(logits, axis=0, keepdims=True)

    @pl.when(pl.program_id(0) == 0)
    def _():
        gm_out[...] = jnp.full((8, NH), -1e30, jnp.float32)

    gm_out[...] = jnp.maximum(gm_out[...], bm)


def _edge_b(ef, ps, pd, pk, pv, eww, ewb):
    eb = lambda w: pl.BlockSpec((EB, w), lambda i: (i, 0))
    full = lambda s: pl.BlockSpec(s, lambda i: (0,) * len(s))

    def wspecs(outw):
        return [full((84, HID)), full((HID, HID)), full((HID, HID)),
                full((1, HID)), full((1, HID)), full((1, HID)),
                full((HID, outw)), full((1, outw))]

    def wargs(p):
        w1t = p['W1'].T.astype(jnp.bfloat16)
        return [w1t[0:84], w1t[84:212], w1t[212:340],
                p['b1'].reshape(1, -1), p['g1'].reshape(1, -1),
                p['be1'].reshape(1, -1), p['W2'].T.astype(jnp.bfloat16),
                p['b2'].reshape(1, -1)]

    return pl.pallas_call(
        _edge_body_b,
        grid=(NEB,),
        in_specs=[full((1, NG)), eb(EF), eb(GW), eb(GW)]
                 + wspecs(HID) + wspecs(NH) + [full((1, RF)), full((1, 1))],
        out_specs=[eb(NH), eb(NH), full((8, NH))],
        out_shape=[jax.ShapeDtypeStruct((E, NH), jnp.float32),
                   jax.ShapeDtypeStruct((E, NH), jnp.float32),
                   jax.ShapeDtypeStruct((8, NH), jnp.float32)],
    )(jnp.asarray(_OFFSET).reshape(1, NG), ef, ps, pd,
      *wargs(pk), *wargs(pv), eww.reshape(1, RF), ewb.reshape(1, 1))


def _payload_body_b(l_ref, w_ref, ps_ref, pd_ref, gm_ref, o_ref):
    gm = jnp.max(gm_ref[...], axis=0, keepdims=True)
    ex = jnp.exp(l_ref[...] - gm)
    p = ex * w_ref[...]
    rel = (lax.bitcast_convert_type(pd_ref[...], jnp.float32)
           - lax.bitcast_convert_type(ps_ref[...], jnp.float32))
    o_ref[...] = jnp.concatenate(
        [p * rel[:, 0:1], p * rel[:, 1:2], p * rel[:, 2:3], ex,
         jnp.zeros_like(rel[:, :64])], axis=1)


def _payload_b(logits, w2, ps, pd, gmax):
    return pl.pallas_call(
        _payload_body_b,
        grid=(NEB,),
        in_specs=[pl.BlockSpec((EB, NH), lambda i: (i, 0)),
                  pl.BlockSpec((EB, NH), lambda i: (i, 0)),
                  pl.BlockSpec((EB, XP), lambda i: (i, 1)),
                  pl.BlockSpec((EB, XP), lambda i: (i, 1)),
                  pl.BlockSpec((8, NH), lambda i: (0, 0))],
        out_specs=pl.BlockSpec((EB, PAY), lambda i: (i, 0)),
        out_shape=jax.ShapeDtypeStruct((E, PAY), jnp.float32),
    )(logits, w2, ps, pd, gmax)


def _final_body_b(a0_ref, a1_ref, xp_ref, m_ref, o_ref):
    a0 = a0_ref[...]
    a1 = a1_ref[...]
    dn = a0[:, 48:64] + a1[:, 48:64]
    dns = jnp.where(dn == 0.0, 1.0, dn)
    outs = []
    for c in range(3):
        nc = a0[:, c * NH:(c + 1) * NH] + a1[:, c * NH:(c + 1) * NH]
        outs.append(jnp.sum(nc / dns, axis=1, keepdims=True) * (1.0 / NH))
    delta = jnp.concatenate(outs + [jnp.zeros_like(a0[:, :125])], axis=1)
    o_ref[...] = xp_ref[...] + delta * m_ref[...]


def _final_b(acc0, acc1, x_pad, mask):
    return pl.pallas_call(
        _final_body_b,
        grid=(NNB,),
        in_specs=[pl.BlockSpec((NB, PAY), lambda i: (i, 0)),
                  pl.BlockSpec((NB, PAY), lambda i: (i, 0)),
                  pl.BlockSpec((NB, XP), lambda i: (i, 0)),
                  pl.BlockSpec((NB, 1), lambda i: (i, 0))],
        out_specs=pl.BlockSpec((NB, XP), lambda i: (i, 0)),
        out_shape=jax.ShapeDtypeStruct((N, XP), jnp.float32),
    )(acc0, acc1, x_pad, mask)


def kernel(h, x, edge_attr, edge_index, mask_ligand, params):
    src = jnp.asarray(edge_index[0], jnp.int32)
    dst = jnp.asarray(edge_index[1], jnp.int32)
    x_pad = jnp.pad(x, ((0, 0), (0, XP - 3)))
    x_i = lax.bitcast_convert_type(x_pad, jnp.int32)
    zeros = jnp.zeros((NP, PAY), jnp.float32)
    p = params

    q = _node_mlp(p['hq'], h)
    tk1 = jnp.concatenate([_pack2bf(h, q), x_i], axis=1)
    ps, pd = _gather2i(src, dst, tk1, tk1)
    v_ew, logits, gmax = _edge_a(edge_attr, ps, pd,
                                 p['hk'], p['hv'],
                                 p['ew_x2h_W'], p['ew_x2h_b'])
    num, den = _payload_a(logits, v_ew, gmax)
    n0, n1 = _scatter128(num, dst, zeros)
    d0, d1 = _scatter128(den, dst, zeros)
    new_h = _combine_a(n0[:N], n1[:N], d0[:N], d1[:N], h, p['node_out'])

    q2 = _node_mlp(p['xq'], new_h)
    tk2 = jnp.concatenate([_pack2bf(new_h, q2), x_i], axis=1)
    ps2, pd2 = _gather2i(src, dst, tk2, tk2)
    w2, logits2, gmax2 = _edge_b(edge_attr, ps2, pd2,
                                 p['xk'], p['xv'],
                                 p['ew_h2x_W'], p['ew_h2x_b'])
    pay2 = _payload_b(logits2, w2, ps2, pd2, gmax2)
    b0, b1 = _scatter128(pay2, dst, zeros)
    x_out = _final_b(b0[:N], b1[:N], x_pad, mask_ligand.reshape(-1, 1))

    return (new_h, x_out[:, :3])


# speedup vs baseline: 1.0823x; 1.0823x over previous
import functools

import jax
import jax.numpy as jnp
import numpy as np
from jax import lax
from jax.experimental import pallas as pl
from jax.experimental.pallas import tpu as pltpu
from jax.experimental.pallas import tpu_sc as plsc

N = 10000
E = 320000
HID = 128
NH = 16
HD = 8
NG = 20
EF = 4
RF = 80

_OFFSET = np.array([0, 1, 1.25, 1.5, 1.75, 2, 2.25, 2.5, 2.75, 3, 3.5, 4,
                    4.5, 5, 5.5, 6, 7, 8, 9, 10], dtype=np.float32)
_COEFF = -0.5
_INV_SQRT_HD = 1.0 / np.sqrt(HD)

EB = 4000
NEB = E // EB
NB = 2000
NNB = N // NB

NW = 32
EPW = E // NW
GC = 80
NCH = EPW // GC
NP = 10240
ZR = NP // 16

PAY = 128
XP = 128
GW = 256


def _ln(xv, g, b):
    mu = jnp.mean(xv, axis=-1, keepdims=True)
    c = xv - mu
    v = jnp.mean(c * c, axis=-1, keepdims=True)
    return c * lax.rsqrt(v + 1e-5) * g + b


def _seg_matrix(dtype=jnp.float32):
    r = lax.broadcasted_iota(jnp.int32, (HID, NH), 0)
    c = lax.broadcasted_iota(jnp.int32, (HID, NH), 1)
    return (r // HD == c).astype(dtype)


def _expand_matrix(dtype=jnp.float32):
    r = lax.broadcasted_iota(jnp.int32, (NH, HID), 0)
    c = lax.broadcasted_iota(jnp.int32, (NH, HID), 1)
    return (c // HD == r).astype(dtype)


def _node_mlp_body(h_ref, w1t, b1, g1, be1, w2t, b2, o_ref):
    x = h_ref[...]
    h1 = jnp.dot(x, w1t[...], preferred_element_type=jnp.float32) + b1[...]
    h1 = jnp.maximum(_ln(h1, g1[...], be1[...]), 0.0)
    o_ref[...] = jnp.dot(h1, w2t[...], preferred_element_type=jnp.float32) + b2[...]


def _node_mlp(p, h):
    full = lambda s: pl.BlockSpec(s, lambda i: (0, 0))
    return pl.pallas_call(
        _node_mlp_body,
        grid=(NNB,),
        in_specs=[
            pl.BlockSpec((NB, HID), lambda i: (i, 0)),
            full((HID, HID)), full((1, HID)), full((1, HID)), full((1, HID)),
            full((HID, HID)), full((1, HID)),
        ],
        out_specs=pl.BlockSpec((NB, HID), lambda i: (i, 0)),
        out_shape=jax.ShapeDtypeStruct((N, HID), jnp.float32),
    )(h, p['W1'].T, p['b1'].reshape(1, -1), p['g1'].reshape(1, -1),
      p['be1'].reshape(1, -1), p['W2'].T, p['b2'].reshape(1, -1))


def _make_gather(widths, dtype=jnp.float32):
    nt = len(widths)

    def body(*refs):
        idx_hbm = refs[:nt]
        tab_hbm = refs[nt:2 * nt]
        out_hbm = refs[2 * nt:3 * nt]
        scratch = refs[3 * nt:]
        idx_v = scratch[:nt]
        row_v = scratch[nt:2 * nt]
        sem = scratch[2 * nt]
        cid = lax.axis_index("c")
        sid = lax.axis_index("s")
        wid = sid * 2 + cid

        def chunk(i, carry):
            base = wid * EPW + i * GC
            for t in range(nt):
                pltpu.sync_copy(idx_hbm[t].at[pl.ds(base, GC)], idx_v[t].at[0])
            copies = [
                pltpu.async_copy(tab_hbm[t].at[idx_v[t].at[0]], row_v[t], sem)
                for t in range(nt)
            ]
            for c in copies:
                c.wait()
            for t in range(nt):
                pltpu.sync_copy(row_v[t], out_hbm[t].at[pl.ds(base, GC)])
            return carry

        lax.fori_loop(0, NCH, chunk, 0)

    scratch = ([pltpu.VMEM((1, GC), jnp.int32) for _ in range(nt)]
               + [pltpu.VMEM((GC, w), dtype) for w in widths]
               + [pltpu.SemaphoreType.DMA])
    out_type = [jax.ShapeDtypeStruct((E, w), dtype) for w in widths]

    built = {}

    def call(*args):
        if 'k' not in built:
            mesh = plsc.VectorSubcoreMesh(core_axis_name="c",
                                          subcore_axis_name="s")
            built['k'] = functools.partial(
                pl.kernel, mesh=mesh, out_type=out_type,
                scratch_types=scratch)(body)
        return built['k'](*args)

    return call


_gather2i = _make_gather([GW, GW], jnp.int32)


def _pack2bf(a, b):
    au = lax.bitcast_convert_type(a.astype(jnp.bfloat16), jnp.uint16)
    bu = lax.bitcast_convert_type(b.astype(jnp.bfloat16), jnp.uint16)
    w = (au.astype(jnp.uint32) << 16) | bu.astype(jnp.uint32)
    return lax.bitcast_convert_type(w, jnp.int32)


def _unpack_hi(u):
    return lax.bitcast_convert_type(u & jnp.int32(-65536), jnp.float32)


def _unpack_lo(u):
    return lax.bitcast_convert_type(u << 16, jnp.float32)


def _make_scatter(width):
    def body(pay_hbm, didx_hbm, zeros_hbm, out0, out1, idx_v, pay_v, acc):
        cid = lax.axis_index("c")
        sid = lax.axis_index("s")
        wid = sid * 2 + cid
        pltpu.sync_copy(zeros_hbm.at[pl.ds(sid * ZR, ZR)],
                        acc.at[pl.ds(sid * ZR, ZR)])
        plsc.subcore_barrier()

        def chunk(i, carry):
            base = wid * EPW + i * GC
            pltpu.sync_copy(didx_hbm.at[pl.ds(base, GC)], idx_v.at[0])
            pltpu.sync_copy(pay_hbm.at[pl.ds(base, GC)], pay_v)
            pltpu.sync_copy(pay_v, acc.at[idx_v.at[0]], add=True)
            return carry

        lax.fori_loop(0, NCH, chunk, 0)
        plsc.subcore_barrier()

        @pl.when(cid == 0)
        def _():
            pltpu.sync_copy(acc.at[pl.ds(sid * ZR, ZR)],
                            out0.at[pl.ds(sid * ZR, ZR)])

        @pl.when(cid == 1)
        def _():
            pltpu.sync_copy(acc.at[pl.ds(sid * ZR, ZR)],
                            out1.at[pl.ds(sid * ZR, ZR)])

    out_type = [jax.ShapeDtypeStruct((NP, width), jnp.float32)] * 2
    scratch = [pltpu.VMEM((1, GC), jnp.int32),
               pltpu.VMEM((GC, width), jnp.float32),
               pltpu.VMEM_SHARED((NP, width), jnp.float32)]

    built = {}

    def call(*args):
        if 'k' not in built:
            mesh = plsc.VectorSubcoreMesh(core_axis_name="c",
                                          subcore_axis_name="s")
            built['k'] = functools.partial(
                pl.kernel, mesh=mesh, out_type=out_type,
                scratch_types=scratch)(body)
        return built['k'](*args)

    return call


_scatter128 = _make_scatter(PAY)


def _edge_body_a(off_ref, ef_ref, ps_ref, pd_ref,
                 kw1ed, kw1i, kw1j, kb1, kg1, kbe1, kw2t, kb2,
                 vw1ed, vw1i, vw1j, vb1, vg1, vbe1, vw2t, vb2,
                 eww, ewb,
                 v_out, l_out, gm_out):
    ps = ps_ref[...]
    pd = pd_ref[...]
    dx = (lax.bitcast_convert_type(pd[:, HID:], jnp.float32)
          - lax.bitcast_convert_type(ps[:, HID:], jnp.float32))
    dist = jnp.sqrt(jnp.sum(dx * dx, axis=1, keepdims=True))
    off = off_ref[...]
    g = jnp.exp(_COEFF * (dist - off) ** 2)
    ef = ef_ref[...]
    df = jnp.concatenate([ef[:, i:i + 1] * g for i in range(EF)], axis=1)
    edf = jnp.concatenate([ef, df], axis=1)
    hs = _unpack_hi(ps[:, :HID])
    pd = pd[:, :HID]
    hd = _unpack_hi(pd)

    def mlp(w1ed, w1i, w1j, b1, g1, be1, w2t, b2):
        pre = (jnp.dot(edf, w1ed[...], preferred_element_type=jnp.float32)
               + jnp.dot(hd, w1i[...], preferred_element_type=jnp.float32)
               + jnp.dot(hs, w1j[...], preferred_element_type=jnp.float32)
               + b1[...])
        h1 = jnp.maximum(_ln(pre, g1[...], be1[...]), 0.0)
        return jnp.dot(h1, w2t[...], preferred_element_type=jnp.float32) + b2[...]

    k = mlp(kw1ed, kw1i, kw1j, kb1, kg1, kbe1, kw2t, kb2)
    v = mlp(vw1ed, vw1i, vw1j, vb1, vg1, vbe1, vw2t, vb2)
    e_w = jax.nn.sigmoid(jnp.sum(df * eww[...], axis=1, keepdims=True)
                         + ewb[...])
    v_out[...] = v * e_w
    qd = _unpack_lo(pd)
    logits = jnp.dot((qd * k) * _INV_SQRT_HD, _seg_matrix(),
                     preferred_element_type=jnp.float32)
    l_out[...] = logits
    bm = jnp.max(logits, axis=0, keepdims=True)

    @pl.when(pl.program_id(0) == 0)
    def _():
        gm_out[...] = jnp.full((8, NH), -1e30, jnp.float32)

    gm_out[...] = jnp.maximum(gm_out[...], bm)


def _edge_a(ef, ps, pd, pk, pv, eww, ewb):
    eb = lambda w: pl.BlockSpec((EB, w), lambda i: (i, 0))
    full = lambda s: pl.BlockSpec(s, lambda i: (0,) * len(s))

    def wspecs():
        return [full((84, HID)), full((HID, HID)), full((HID, HID)),
                full((1, HID)), full((1, HID)), full((1, HID)),
                full((HID, HID)), full((1, HID))]

    def wargs(p):
        w1t = p['W1'].T
        return [w1t[0:84], w1t[84:212], w1t[212:340],
                p['b1'].reshape(1, -1), p['g1'].reshape(1, -1),
                p['be1'].reshape(1, -1), p['W2'].T, p['b2'].reshape(1, -1)]

    return pl.pallas_call(
        _edge_body_a,
        grid=(NEB,),
        in_specs=[full((1, NG)), eb(EF), eb(GW), eb(GW)]
                 + wspecs() + wspecs() + [full((1, RF)), full((1, 1))],
        out_specs=[eb(HID), eb(NH), full((8, NH))],
        out_shape=[jax.ShapeDtypeStruct((E, HID), jnp.float32),
                   jax.ShapeDtypeStruct((E, NH), jnp.float32),
                   jax.ShapeDtypeStruct((8, NH), jnp.float32)],
    )(jnp.asarray(_OFFSET).reshape(1, NG), ef, ps, pd,
      *wargs(pk), *wargs(pv), eww.reshape(1, RF), ewb.reshape(1, 1))


def _payload_body_a(l_ref, v_ref, gm_ref, n_ref, d_ref):
    gm = jnp.max(gm_ref[...], axis=0, keepdims=True)
    ex = jnp.exp(l_ref[...] - gm)
    exb = jnp.dot(ex, _expand_matrix(), preferred_element_type=jnp.float32)
    n_ref[...] = v_ref[...] * exb
    d_ref[...] = exb


def _payload_a(logits, v_ew, gmax):
    return pl.pallas_call(
        _payload_body_a,
        grid=(NEB,),
        in_specs=[pl.BlockSpec((EB, NH), lambda i: (i, 0)),
                  pl.BlockSpec((EB, HID), lambda i: (i, 0)),
                  pl.BlockSpec((8, NH), lambda i: (0, 0))],
        out_specs=[pl.BlockSpec((EB, PAY), lambda i: (i, 0)),
                   pl.BlockSpec((EB, PAY), lambda i: (i, 0))],
        out_shape=[jax.ShapeDtypeStruct((E, PAY), jnp.float32),
                   jax.ShapeDtypeStruct((E, PAY), jnp.float32)],
    )(logits, v_ew, gmax)


def _combine_body_a(n0_ref, n1_ref, d0_ref, d1_ref, h_ref,
                    w1a, w1b, b1, g1, be1, w2t, b2, o_ref):
    num = n0_ref[...] + n1_ref[...]
    dnb = d0_ref[...] + d1_ref[...]
    attn = num / jnp.where(dnb == 0.0, 1.0, dnb)
    h = h_ref[...]
    pre = (jnp.dot(attn, w1a[...], preferred_element_type=jnp.float32)
           + jnp.dot(h, w1b[...], preferred_element_type=jnp.float32)
           + b1[...])
    h1 = jnp.maximum(_ln(pre, g1[...], be1[...]), 0.0)
    o_ref[...] = (jnp.dot(h1, w2t[...], preferred_element_type=jnp.float32)
                  + b2[...] + h)


def _combine_a(n0, n1, d0, d1, h, p):
    full = lambda s: pl.BlockSpec(s, lambda i: (0, 0))
    nb = lambda w: pl.BlockSpec((NB, w), lambda i: (i, 0))
    w1t = p['W1'].T
    return pl.pallas_call(
        _combine_body_a,
        grid=(NNB,),
        in_specs=[nb(PAY), nb(PAY), nb(PAY), nb(PAY), nb(HID),
                  full((HID, HID)), full((HID, HID)), full((1, HID)),
                  full((1, HID)), full((1, HID)), full((HID, HID)),
                  full((1, HID))],
        out_specs=pl.BlockSpec((NB, HID), lambda i: (i, 0)),
        out_shape=jax.ShapeDtypeStruct((N, HID), jnp.float32),
    )(n0, n1, d0, d1, h, w1t[:HID], w1t[HID:], p['b1'].reshape(1, -1),
      p['g1'].reshape(1, -1), p['be1'].reshape(1, -1), p['W2'].T,
      p['b2'].reshape(1, -1))


def _edge_body_b(off_ref, ef_ref, ps_ref, pd_ref,
                 kw1ed, kw1i, kw1j, kb1, kg1, kbe1, kw2t, kb2,
                 vw1ed, vw1i, vw1j, vb1, vg1, vbe1, vw2t, vb2,
                 eww, ewb,
                 w_out, l_out, gm_out):
    ps = ps_ref[...]
    pd = pd_ref[...]
    dx = (lax.bitcast_convert_type(pd[:, HID:], jnp.float32)
          - lax.bitcast_convert_type(ps[:, HID:], jnp.float32))
    dist = jnp.sqrt(jnp.sum(dx * dx, axis=1, keepdims=True))
    off = off_ref[...]
    g = jnp.exp(_COEFF * (dist - off) ** 2)
    ef = ef_ref[...]
    df = jnp.concatenate([ef[:, i:i + 1] * g for i in range(EF)], axis=1)
    edf = jnp.concatenate([ef, df], axis=1)
    hs = _unpack_hi(ps[:, :HID])
    pd = pd[:, :HID]
    hd = _unpack_hi(pd)

    def mlp(w1ed, w1i, w1j, b1, g1, be1, w2t, b2):
        pre = (jnp.dot(edf, w1ed[...], preferred_element_type=jnp.float32)
               + jnp.dot(hd, w1i[...], preferred_element_type=jnp.float32)
               + jnp.dot(hs, w1j[...], preferred_element_type=jnp.float32)
               + b1[...])
        h1 = jnp.maximum(_ln(pre, g1[...], be1[...]), 0.0)
        return jnp.dot(h1, w2t[...], preferred_element_type=jnp.float32) + b2[...]

    k = mlp(kw1ed, kw1i, kw1j, kb1, kg1, kbe1, kw2t, kb2)
    v2 = mlp(vw1ed, vw1i, vw1j, vb1, vg1, vbe1, vw2t, vb2)
    e_w = jax.nn.sigmoid(jnp.sum(df * eww[...], axis=1, keepdims=True)
                         + ewb[...])
    w_out[...] = v2 * e_w
    qd = _unpack_lo(pd)
    logits = jnp.dot((qd * k) * _INV_SQRT_HD, _seg_matrix(),
                     preferred_element_type=jnp.float32)
    l_out[...] = logits
    bm = jnp.max(logits, axis=0, keepdims=True)

    @pl.when(pl.program_id(0) == 0)
    def _():
        gm_out[...] = jnp.full((8, NH), -1e30, jnp.float32)

    gm_out[...] = jnp.maximum(gm_out[...], bm)


def _edge_b(ef, ps, pd, pk, pv, eww, ewb):
    eb = lambda w: pl.BlockSpec((EB, w), lambda i: (i, 0))
    full = lambda s: pl.BlockSpec(s, lambda i: (0,) * len(s))

    def wspecs(outw):
        return [full((84, HID)), full((HID, HID)), full((HID, HID)),
                full((1, HID)), full((1, HID)), full((1, HID)),
                full((HID, outw)), full((1, outw))]

    def wargs(p):
        w1t = p['W1'].T
        return [w1t[0:84], w1t[84:212], w1t[212:340],
                p['b1'].reshape(1, -1), p['g1'].reshape(1, -1),
                p['be1'].reshape(1, -1), p['W2'].T, p['b2'].reshape(1, -1)]

    return pl.pallas_call(
        _edge_body_b,
        grid=(NEB,),
        in_specs=[full((1, NG)), eb(EF), eb(GW), eb(GW)]
                 + wspecs(HID) + wspecs(NH) + [full((1, RF)), full((1, 1))],
        out_specs=[eb(NH), eb(NH), full((8, NH))],
        out_shape=[jax.ShapeDtypeStruct((E, NH), jnp.float32),
                   jax.ShapeDtypeStruct((E, NH), jnp.float32),
                   jax.ShapeDtypeStruct((8, NH), jnp.float32)],
    )(jnp.asarray(_OFFSET).reshape(1, NG), ef, ps, pd,
      *wargs(pk), *wargs(pv), eww.reshape(1, RF), ewb.reshape(1, 1))


def _payload_body_b(l_ref, w_ref, ps_ref, pd_ref, gm_ref, o_ref):
    gm = jnp.max(gm_ref[...], axis=0, keepdims=True)
    ex = jnp.exp(l_ref[...] - gm)
    p = ex * w_ref[...]
    rel = (lax.bitcast_convert_type(pd_ref[...], jnp.float32)
           - lax.bitcast_convert_type(ps_ref[...], jnp.float32))
    o_ref[...] = jnp.concatenate(
        [p * rel[:, 0:1], p * rel[:, 1:2], p * rel[:, 2:3], ex,
         jnp.zeros_like(rel[:, :64])], axis=1)


def _payload_b(logits, w2, ps, pd, gmax):
    return pl.pallas_call(
        _payload_body_b,
        grid=(NEB,),
        in_specs=[pl.BlockSpec((EB, NH), lambda i: (i, 0)),
                  pl.BlockSpec((EB, NH), lambda i: (i, 0)),
                  pl.BlockSpec((EB, XP), lambda i: (i, 1)),
                  pl.BlockSpec((EB, XP), lambda i: (i, 1)),
                  pl.BlockSpec((8, NH), lambda i: (0, 0))],
        out_specs=pl.BlockSpec((EB, PAY), lambda i: (i, 0)),
        out_shape=jax.ShapeDtypeStruct((E, PAY), jnp.float32),
    )(logits, w2, ps, pd, gmax)


def _final_body_b(a0_ref, a1_ref, xp_ref, m_ref, o_ref):
    a0 = a0_ref[...]
    a1 = a1_ref[...]
    dn = a0[:, 48:64] + a1[:, 48:64]
    dns = jnp.where(dn == 0.0, 1.0, dn)
    outs = []
    for c in range(3):
        nc = a0[:, c * NH:(c + 1) * NH] + a1[:, c * NH:(c + 1) * NH]
        outs.append(jnp.sum(nc / dns, axis=1, keepdims=True) * (1.0 / NH))
    delta = jnp.concatenate(outs + [jnp.zeros_like(a0[:, :125])], axis=1)
    o_ref[...] = xp_ref[...] + delta * m_ref[...]


def _final_b(acc0, acc1, x_pad, mask):
    return pl.pallas_call(
        _final_body_b,
        grid=(NNB,),
        in_specs=[pl.BlockSpec((NB, PAY), lambda i: (i, 0)),
                  pl.BlockSpec((NB, PAY), lambda i: (i, 0)),
                  pl.BlockSpec((NB, XP), lambda i: (i, 0)),
                  pl.BlockSpec((NB, 1), lambda i: (i, 0))],
        out_specs=pl.BlockSpec((NB, XP), lambda i: (i, 0)),
        out_shape=jax.ShapeDtypeStruct((N, XP), jnp.float32),
    )(acc0, acc1, x_pad, mask)


def kernel(h, x, edge_attr, edge_index, mask_ligand, params):
    src = jnp.asarray(edge_index[0], jnp.int32)
    dst = jnp.asarray(edge_index[1], jnp.int32)
    x_pad = jnp.pad(x, ((0, 0), (0, XP - 3)))
    x_i = lax.bitcast_convert_type(x_pad, jnp.int32)
    zeros = jnp.zeros((NP, PAY), jnp.float32)
    p = params

    q = _node_mlp(p['hq'], h)
    tk1 = jnp.concatenate([_pack2bf(h, q), x_i], axis=1)
    ps, pd = _gather2i(src, dst, tk1, tk1)
    v_ew, logits, gmax = _edge_a(edge_attr, ps, pd,
                                 p['hk'], p['hv'],
                                 p['ew_x2h_W'], p['ew_x2h_b'])
    num, den = _payload_a(logits, v_ew, gmax)
    n0, n1 = _scatter128(num, dst, zeros)
    d0, d1 = _scatter128(den, dst, zeros)
    new_h = _combine_a(n0[:N], n1[:N], d0[:N], d1[:N], h, p['node_out'])

    q2 = _node_mlp(p['xq'], new_h)
    tk2 = jnp.concatenate([_pack2bf(new_h, q2), x_i], axis=1)
    ps2, pd2 = _gather2i(src, dst, tk2, tk2)
    w2, logits2, gmax2 = _edge_b(edge_attr, ps2, pd2,
                                 p['xk'], p['xv'],
                                 p['ew_h2x_W'], p['ew_h2x_b'])
    pay2 = _payload_b(logits2, w2, ps2, pd2, gmax2)
    b0, b1 = _scatter128(pay2, dst, zeros)
    x_out = _final_b(b0[:N], b1[:N], x_pad, mask_ligand.reshape(-1, 1))

    return (new_h, x_out[:, :3])


# speedup vs baseline: 1.0924x; 1.0093x over previous
import functools

import jax
import jax.numpy as jnp
import numpy as np
from jax import lax
from jax.experimental import pallas as pl
from jax.experimental.pallas import tpu as pltpu
from jax.experimental.pallas import tpu_sc as plsc

N = 10000
E = 320000
HID = 128
NH = 16
HD = 8
NG = 20
EF = 4
RF = 80

_OFFSET = np.array([0, 1, 1.25, 1.5, 1.75, 2, 2.25, 2.5, 2.75, 3, 3.5, 4,
                    4.5, 5, 5.5, 6, 7, 8, 9, 10], dtype=np.float32)
_COEFF = -0.5
_INV_SQRT_HD = 1.0 / np.sqrt(HD)

EB = 5000
NEB = E // EB
NB = 2000
NNB = N // NB

NW = 32
EPW = E // NW
GC = 80
NCH = EPW // GC
NP = 10240
ZR = NP // 16

PAY = 128
XP = 128
GW = 256


def _ln(xv, g, b):
    mu = jnp.mean(xv, axis=-1, keepdims=True)
    c = xv - mu
    v = jnp.mean(c * c, axis=-1, keepdims=True)
    return c * lax.rsqrt(v + 1e-5) * g + b


def _seg_matrix(dtype=jnp.float32):
    r = lax.broadcasted_iota(jnp.int32, (HID, NH), 0)
    c = lax.broadcasted_iota(jnp.int32, (HID, NH), 1)
    return (r // HD == c).astype(dtype)


def _expand_matrix(dtype=jnp.float32):
    r = lax.broadcasted_iota(jnp.int32, (NH, HID), 0)
    c = lax.broadcasted_iota(jnp.int32, (NH, HID), 1)
    return (c // HD == r).astype(dtype)


def _node_mlp_body(h_ref, w1t, b1, g1, be1, w2t, b2, o_ref):
    x = h_ref[...]
    h1 = jnp.dot(x, w1t[...], preferred_element_type=jnp.float32) + b1[...]
    h1 = jnp.maximum(_ln(h1, g1[...], be1[...]), 0.0)
    o_ref[...] = jnp.dot(h1, w2t[...], preferred_element_type=jnp.float32) + b2[...]


def _node_mlp(p, h):
    full = lambda s: pl.BlockSpec(s, lambda i: (0, 0))
    return pl.pallas_call(
        _node_mlp_body,
        grid=(NNB,),
        in_specs=[
            pl.BlockSpec((NB, HID), lambda i: (i, 0)),
            full((HID, HID)), full((1, HID)), full((1, HID)), full((1, HID)),
            full((HID, HID)), full((1, HID)),
        ],
        out_specs=pl.BlockSpec((NB, HID), lambda i: (i, 0)),
        out_shape=jax.ShapeDtypeStruct((N, HID), jnp.float32),
    )(h, p['W1'].T, p['b1'].reshape(1, -1), p['g1'].reshape(1, -1),
      p['be1'].reshape(1, -1), p['W2'].T, p['b2'].reshape(1, -1))


def _make_gather(widths, dtype=jnp.float32):
    nt = len(widths)

    def body(*refs):
        idx_hbm = refs[:nt]
        tab_hbm = refs[nt:2 * nt]
        out_hbm = refs[2 * nt:3 * nt]
        scratch = refs[3 * nt:]
        idx_v = scratch[:nt]
        row_v = scratch[nt:2 * nt]
        sem = scratch[2 * nt]
        cid = lax.axis_index("c")
        sid = lax.axis_index("s")
        wid = sid * 2 + cid

        def chunk(i, carry):
            base = wid * EPW + i * GC
            for t in range(nt):
                pltpu.sync_copy(idx_hbm[t].at[pl.ds(base, GC)], idx_v[t].at[0])
            copies = [
                pltpu.async_copy(tab_hbm[t].at[idx_v[t].at[0]], row_v[t], sem)
                for t in range(nt)
            ]
            for c in copies:
                c.wait()
            for t in range(nt):
                pltpu.sync_copy(row_v[t], out_hbm[t].at[pl.ds(base, GC)])
            return carry

        lax.fori_loop(0, NCH, chunk, 0)

    scratch = ([pltpu.VMEM((1, GC), jnp.int32) for _ in range(nt)]
               + [pltpu.VMEM((GC, w), dtype) for w in widths]
               + [pltpu.SemaphoreType.DMA])
    out_type = [jax.ShapeDtypeStruct((E, w), dtype) for w in widths]

    built = {}

    def call(*args):
        if 'k' not in built:
            mesh = plsc.VectorSubcoreMesh(core_axis_name="c",
                                          subcore_axis_name="s")
            built['k'] = functools.partial(
                pl.kernel, mesh=mesh, out_type=out_type,
                scratch_types=scratch)(body)
        return built['k'](*args)

    return call


_gather2i = _make_gather([GW, GW], jnp.int32)


def _pack2bf(a, b):
    au = lax.bitcast_convert_type(a.astype(jnp.bfloat16), jnp.uint16)
    bu = lax.bitcast_convert_type(b.astype(jnp.bfloat16), jnp.uint16)
    w = (au.astype(jnp.uint32) << 16) | bu.astype(jnp.uint32)
    return lax.bitcast_convert_type(w, jnp.int32)


def _unpack_hi(u):
    return lax.bitcast_convert_type(u & jnp.int32(-65536), jnp.float32)


def _unpack_lo(u):
    return lax.bitcast_convert_type(u << 16, jnp.float32)


def _make_scatter(width):
    def body(pay_hbm, didx_hbm, zeros_hbm, out0, out1, idx_v, pay_v, acc):
        cid = lax.axis_index("c")
        sid = lax.axis_index("s")
        wid = sid * 2 + cid
        pltpu.sync_copy(zeros_hbm.at[pl.ds(sid * ZR, ZR)],
                        acc.at[pl.ds(sid * ZR, ZR)])
        plsc.subcore_barrier()

        def chunk(i, carry):
            base = wid * EPW + i * GC
            pltpu.sync_copy(didx_hbm.at[pl.ds(base, GC)], idx_v.at[0])
            pltpu.sync_copy(pay_hbm.at[pl.ds(base, GC)], pay_v)
            pltpu.sync_copy(pay_v, acc.at[idx_v.at[0]], add=True)
            return carry

        lax.fori_loop(0, NCH, chunk, 0)
        plsc.subcore_barrier()

        @pl.when(cid == 0)
        def _():
            pltpu.sync_copy(acc.at[pl.ds(sid * ZR, ZR)],
                            out0.at[pl.ds(sid * ZR, ZR)])

        @pl.when(cid == 1)
        def _():
            pltpu.sync_copy(acc.at[pl.ds(sid * ZR, ZR)],
                            out1.at[pl.ds(sid * ZR, ZR)])

    out_type = [jax.ShapeDtypeStruct((NP, width), jnp.float32)] * 2
    scratch = [pltpu.VMEM((1, GC), jnp.int32),
               pltpu.VMEM((GC, width), jnp.float32),
               pltpu.VMEM_SHARED((NP, width), jnp.float32)]

    built = {}

    def call(*args):
        if 'k' not in built:
            mesh = plsc.VectorSubcoreMesh(core_axis_name="c",
                                          subcore_axis_name="s")
            built['k'] = functools.partial(
                pl.kernel, mesh=mesh, out_type=out_type,
                scratch_types=scratch)(body)
        return built['k'](*args)

    return call


_scatter128 = _make_scatter(PAY)


def _edge_body_a(off_ref, ef_ref, ps_ref, pd_ref,
                 kw1ed, kw1i, kw1j, kb1, kg1, kbe1, kw2t, kb2,
                 vw1ed, vw1i, vw1j, vb1, vg1, vbe1, vw2t, vb2,
                 eww, ewb,
                 v_out, l_out, gm_out):
    ps = ps_ref[...]
    pd = pd_ref[...]
    dx = (lax.bitcast_convert_type(pd[:, HID:], jnp.float32)
          - lax.bitcast_convert_type(ps[:, HID:], jnp.float32))
    dist = jnp.sqrt(jnp.sum(dx * dx, axis=1, keepdims=True))
    off = off_ref[...]
    g = jnp.exp(_COEFF * (dist - off) ** 2)
    ef = ef_ref[...]
    df = jnp.concatenate([ef[:, i:i + 1] * g for i in range(EF)], axis=1)
    edf = jnp.concatenate([ef, df], axis=1)
    hs = _unpack_hi(ps[:, :HID])
    pd = pd[:, :HID]
    hd = _unpack_hi(pd)

    def mlp(w1ed, w1i, w1j, b1, g1, be1, w2t, b2):
        pre = (jnp.dot(edf, w1ed[...], preferred_element_type=jnp.float32)
               + jnp.dot(hd, w1i[...], preferred_element_type=jnp.float32)
               + jnp.dot(hs, w1j[...], preferred_element_type=jnp.float32)
               + b1[...])
        h1 = jnp.maximum(_ln(pre, g1[...], be1[...]), 0.0)
        return jnp.dot(h1, w2t[...], preferred_element_type=jnp.float32) + b2[...]

    k = mlp(kw1ed, kw1i, kw1j, kb1, kg1, kbe1, kw2t, kb2)
    v = mlp(vw1ed, vw1i, vw1j, vb1, vg1, vbe1, vw2t, vb2)
    e_w = jax.nn.sigmoid(jnp.sum(df * eww[...], axis=1, keepdims=True)
                         + ewb[...])
    v_out[...] = v * e_w
    qd = _unpack_lo(pd)
    logits = jnp.dot((qd * k) * _INV_SQRT_HD, _seg_matrix(),
                     preferred_element_type=jnp.float32)
    l_out[...] = logits
    bm = jnp.max(logits, axis=0, keepdims=True)

    @pl.when(pl.program_id(0) == 0)
    def _():
        gm_out[...] = jnp.full((8, NH), -1e30, jnp.float32)

    gm_out[...] = jnp.maximum(gm_out[...], bm)


def _edge_a(ef, ps, pd, pk, pv, eww, ewb):
    eb = lambda w: pl.BlockSpec((EB, w), lambda i: (i, 0))
    full = lambda s: pl.BlockSpec(s, lambda i: (0,) * len(s))

    def wspecs():
        return [full((84, HID)), full((HID, HID)), full((HID, HID)),
                full((1, HID)), full((1, HID)), full((1, HID)),
                full((HID, HID)), full((1, HID))]

    def wargs(p):
        w1t = p['W1'].T
        return [w1t[0:84], w1t[84:212], w1t[212:340],
                p['b1'].reshape(1, -1), p['g1'].reshape(1, -1),
                p['be1'].reshape(1, -1), p['W2'].T, p['b2'].reshape(1, -1)]

    return pl.pallas_call(
        _edge_body_a,
        grid=(NEB,),
        in_specs=[full((1, NG)), eb(EF), eb(GW), eb(GW)]
                 + wspecs() + wspecs() + [full((1, RF)), full((1, 1))],
        out_specs=[eb(HID), eb(NH), full((8, NH))],
        out_shape=[jax.ShapeDtypeStruct((E, HID), jnp.float32),
                   jax.ShapeDtypeStruct((E, NH), jnp.float32),
                   jax.ShapeDtypeStruct((8, NH), jnp.float32)],
    )(jnp.asarray(_OFFSET).reshape(1, NG), ef, ps, pd,
      *wargs(pk), *wargs(pv), eww.reshape(1, RF), ewb.reshape(1, 1))


def _payload_body_a(l_ref, v_ref, gm_ref, n_ref, d_ref):
    gm = jnp.max(gm_ref[...], axis=0, keepdims=True)
    ex = jnp.exp(l_ref[...] - gm)
    exb = jnp.dot(ex, _expand_matrix(), preferred_element_type=jnp.float32)
    n_ref[...] = v_ref[...] * exb
    d_ref[...] = exb


def _payload_a(logits, v_ew, gmax):
    return pl.pallas_call(
        _payload_body_a,
        grid=(NEB,),
        in_specs=[pl.BlockSpec((EB, NH), lambda i: (i, 0)),
                  pl.BlockSpec((EB, HID), lambda i: (i, 0)),
                  pl.BlockSpec((8, NH), lambda i: (0, 0))],
        out_specs=[pl.BlockSpec((EB, PAY), lambda i: (i, 0)),
                   pl.BlockSpec((EB, PAY), lambda i: (i, 0))],
        out_shape=[jax.ShapeDtypeStruct((E, PAY), jnp.float32),
                   jax.ShapeDtypeStruct((E, PAY), jnp.float32)],
    )(logits, v_ew, gmax)


def _combine_body_a(n0_ref, n1_ref, d0_ref, d1_ref, h_ref,
                    w1a, w1b, b1, g1, be1, w2t, b2, o_ref):
    num = n0_ref[...] + n1_ref[...]
    dnb = d0_ref[...] + d1_ref[...]
    attn = num / jnp.where(dnb == 0.0, 1.0, dnb)
    h = h_ref[...]
    pre = (jnp.dot(attn, w1a[...], preferred_element_type=jnp.float32)
           + jnp.dot(h, w1b[...], preferred_element_type=jnp.float32)
           + b1[...])
    h1 = jnp.maximum(_ln(pre, g1[...], be1[...]), 0.0)
    o_ref[...] = (jnp.dot(h1, w2t[...], preferred_element_type=jnp.float32)
                  + b2[...] + h)


def _combine_a(n0, n1, d0, d1, h, p):
    full = lambda s: pl.BlockSpec(s, lambda i: (0, 0))
    nb = lambda w: pl.BlockSpec((NB, w), lambda i: (i, 0))
    w1t = p['W1'].T
    return pl.pallas_call(
        _combine_body_a,
        grid=(NNB,),
        in_specs=[nb(PAY), nb(PAY), nb(PAY), nb(PAY), nb(HID),
                  full((HID, HID)), full((HID, HID)), full((1, HID)),
                  full((1, HID)), full((1, HID)), full((HID, HID)),
                  full((1, HID))],
        out_specs=pl.BlockSpec((NB, HID), lambda i: (i, 0)),
        out_shape=jax.ShapeDtypeStruct((N, HID), jnp.float32),
    )(n0, n1, d0, d1, h, w1t[:HID], w1t[HID:], p['b1'].reshape(1, -1),
      p['g1'].reshape(1, -1), p['be1'].reshape(1, -1), p['W2'].T,
      p['b2'].reshape(1, -1))


def _edge_body_b(off_ref, ef_ref, ps_ref, pd_ref,
                 kw1ed, kw1i, kw1j, kb1, kg1, kbe1, kw2t, kb2,
                 vw1ed, vw1i, vw1j, vb1, vg1, vbe1, vw2t, vb2,
                 eww, ewb,
                 w_out, l_out, gm_out):
    ps = ps_ref[...]
    pd = pd_ref[...]
    dx = (lax.bitcast_convert_type(pd[:, HID:], jnp.float32)
          - lax.bitcast_convert_type(ps[:, HID:], jnp.float32))
    dist = jnp.sqrt(jnp.sum(dx * dx, axis=1, keepdims=True))
    off = off_ref[...]
    g = jnp.exp(_COEFF * (dist - off) ** 2)
    ef = ef_ref[...]
    df = jnp.concatenate([ef[:, i:i + 1] * g for i in range(EF)], axis=1)
    edf = jnp.concatenate([ef, df], axis=1)
    hs = _unpack_hi(ps[:, :HID])
    pd = pd[:, :HID]
    hd = _unpack_hi(pd)

    def mlp(w1ed, w1i, w1j, b1, g1, be1, w2t, b2):
        pre = (jnp.dot(edf, w1ed[...], preferred_element_type=jnp.float32)
               + jnp.dot(hd, w1i[...], preferred_element_type=jnp.float32)
               + jnp.dot(hs, w1j[...], preferred_element_type=jnp.float32)
               + b1[...])
        h1 = jnp.maximum(_ln(pre, g1[...], be1[...]), 0.0)
        return jnp.dot(h1, w2t[...], preferred_element_type=jnp.float32) + b2[...]

    k = mlp(kw1ed, kw1i, kw1j, kb1, kg1, kbe1, kw2t, kb2)
    v2 = mlp(vw1ed, vw1i, vw1j, vb1, vg1, vbe1, vw2t, vb2)
    e_w = jax.nn.sigmoid(jnp.sum(df * eww[...], axis=1, keepdims=True)
                         + ewb[...])
    w_out[...] = v2 * e_w
    qd = _unpack_lo(pd)
    logits = jnp.dot((qd * k) * _INV_SQRT_HD, _seg_matrix(),
                     preferred_element_type=jnp.float32)
    l_out[...] = logits
    bm = jnp.max(logits, axis=0, keepdims=True)

    @pl.when(pl.program_id(0) == 0)
    def _():
        gm_out[...] = jnp.full((8, NH), -1e30, jnp.float32)

    gm_out[...] = jnp.maximum(gm_out[...], bm)


def _edge_b(ef, ps, pd, pk, pv, eww, ewb):
    eb = lambda w: pl.BlockSpec((EB, w), lambda i: (i, 0))
    full = lambda s: pl.BlockSpec(s, lambda i: (0,) * len(s))

    def wspecs(outw):
        return [full((84, HID)), full((HID, HID)), full((HID, HID)),
                full((1, HID)), full((1, HID)), full((1, HID)),
                full((HID, outw)), full((1, outw))]

    def wargs(p):
        w1t = p['W1'].T
        return [w1t[0:84], w1t[84:212], w1t[212:340],
                p['b1'].reshape(1, -1), p['g1'].reshape(1, -1),
                p['be1'].reshape(1, -1), p['W2'].T, p['b2'].reshape(1, -1)]

    return pl.pallas_call(
        _edge_body_b,
        grid=(NEB,),
        in_specs=[full((1, NG)), eb(EF), eb(GW), eb(GW)]
                 + wspecs(HID) + wspecs(NH) + [full((1, RF)), full((1, 1))],
        out_specs=[eb(NH), eb(NH), full((8, NH))],
        out_shape=[jax.ShapeDtypeStruct((E, NH), jnp.float32),
                   jax.ShapeDtypeStruct((E, NH), jnp.float32),
                   jax.ShapeDtypeStruct((8, NH), jnp.float32)],
    )(jnp.asarray(_OFFSET).reshape(1, NG), ef, ps, pd,
      *wargs(pk), *wargs(pv), eww.reshape(1, RF), ewb.reshape(1, 1))


def _payload_body_b(l_ref, w_ref, ps_ref, pd_ref, gm_ref, o_ref):
    gm = jnp.max(gm_ref[...], axis=0, keepdims=True)
    ex = jnp.exp(l_ref[...] - gm)
    p = ex * w_ref[...]
    rel = (lax.bitcast_convert_type(pd_ref[...], jnp.float32)
           - lax.bitcast_convert_type(ps_ref[...], jnp.float32))
    o_ref[...] = jnp.concatenate(
        [p * rel[:, 0:1], p * rel[:, 1:2], p * rel[:, 2:3], ex,
         jnp.zeros_like(rel[:, :64])], axis=1)


def _payload_b(logits, w2, ps, pd, gmax):
    return pl.pallas_call(
        _payload_body_b,
        grid=(NEB,),
        in_specs=[pl.BlockSpec((EB, NH), lambda i: (i, 0)),
                  pl.BlockSpec((EB, NH), lambda i: (i, 0)),
                  pl.BlockSpec((EB, XP), lambda i: (i, 1)),
                  pl.BlockSpec((EB, XP), lambda i: (i, 1)),
                  pl.BlockSpec((8, NH), lambda i: (0, 0))],
        out_specs=pl.BlockSpec((EB, PAY), lambda i: (i, 0)),
        out_shape=jax.ShapeDtypeStruct((E, PAY), jnp.float32),
    )(logits, w2, ps, pd, gmax)


def _final_body_b(a0_ref, a1_ref, xp_ref, m_ref, o_ref):
    a0 = a0_ref[...]
    a1 = a1_ref[...]
    dn = a0[:, 48:64] + a1[:, 48:64]
    dns = jnp.where(dn == 0.0, 1.0, dn)
    outs = []
    for c in range(3):
        nc = a0[:, c * NH:(c + 1) * NH] + a1[:, c * NH:(c + 1) * NH]
        outs.append(jnp.sum(nc / dns, axis=1, keepdims=True) * (1.0 / NH))
    delta = jnp.concatenate(outs + [jnp.zeros_like(a0[:, :125])], axis=1)
    o_ref[...] = xp_ref[...] + delta * m_ref[...]


def _final_b(acc0, acc1, x_pad, mask):
    return pl.pallas_call(
        _final_body_b,
        grid=(NNB,),
        in_specs=[pl.BlockSpec((NB, PAY), lambda i: (i, 0)),
                  pl.BlockSpec((NB, PAY), lambda i: (i, 0)),
                  pl.BlockSpec((NB, XP), lambda i: (i, 0)),
                  pl.BlockSpec((NB, 1), lambda i: (i, 0))],
        out_specs=pl.BlockSpec((NB, XP), lambda i: (i, 0)),
        out_shape=jax.ShapeDtypeStruct((N, XP), jnp.float32),
    )(acc0, acc1, x_pad, mask)


def kernel(h, x, edge_attr, edge_index, mask_ligand, params):
    src = jnp.asarray(edge_index[0], jnp.int32)
    dst = jnp.asarray(edge_index[1], jnp.int32)
    x_pad = jnp.pad(x, ((0, 0), (0, XP - 3)))
    x_i = lax.bitcast_convert_type(x_pad, jnp.int32)
    zeros = jnp.zeros((NP, PAY), jnp.float32)
    p = params

    q = _node_mlp(p['hq'], h)
    tk1 = jnp.concatenate([_pack2bf(h, q), x_i], axis=1)
    ps, pd = _gather2i(src, dst, tk1, tk1)
    v_ew, logits, gmax = _edge_a(edge_attr, ps, pd,
                                 p['hk'], p['hv'],
                                 p['ew_x2h_W'], p['ew_x2h_b'])
    num, den = _payload_a(logits, v_ew, gmax)
    n0, n1 = _scatter128(num, dst, zeros)
    d0, d1 = _scatter128(den, dst, zeros)
    new_h = _combine_a(n0[:N], n1[:N], d0[:N], d1[:N], h, p['node_out'])

    q2 = _node_mlp(p['xq'], new_h)
    tk2 = jnp.concatenate([_pack2bf(new_h, q2), x_i], axis=1)
    ps2, pd2 = _gather2i(src, dst, tk2, tk2)
    w2, logits2, gmax2 = _edge_b(edge_attr, ps2, pd2,
                                 p['xk'], p['xv'],
                                 p['ew_h2x_W'], p['ew_h2x_b'])
    pay2 = _payload_b(logits2, w2, ps2, pd2, gmax2)
    b0, b1 = _scatter128(pay2, dst, zeros)
    x_out = _final_b(b0[:N], b1[:N], x_pad, mask_ligand.reshape(-1, 1))

    return (new_h, x_out[:, :3])


# speedup vs baseline: 1.1655x; 1.0669x over previous
import functools

import jax
import jax.numpy as jnp
import numpy as np
from jax import lax
from jax.experimental import pallas as pl
from jax.experimental.pallas import tpu as pltpu
from jax.experimental.pallas import tpu_sc as plsc

N = 10000
E = 320000
HID = 128
NH = 16
HD = 8
NG = 20
EF = 4
RF = 80

_OFFSET = np.array([0, 1, 1.25, 1.5, 1.75, 2, 2.25, 2.5, 2.75, 3, 3.5, 4,
                    4.5, 5, 5.5, 6, 7, 8, 9, 10], dtype=np.float32)
_COEFF = -0.5
_INV_SQRT_HD = 1.0 / np.sqrt(HD)

EB = 5000
NEB = E // EB
NB = 2000
NNB = N // NB

NW = 32
EPW = E // NW
GC = 80
NCH = EPW // GC
NP = 10240
ZR = NP // 16

PAY = 128
XP = 128
GW = 256


def _ln(xv, g, b):
    mu = jnp.mean(xv, axis=-1, keepdims=True)
    c = xv - mu
    v = jnp.mean(c * c, axis=-1, keepdims=True)
    return c * lax.rsqrt(v + 1e-5) * g + b


def _seg_matrix(dtype=jnp.float32):
    r = lax.broadcasted_iota(jnp.int32, (HID, NH), 0)
    c = lax.broadcasted_iota(jnp.int32, (HID, NH), 1)
    return (r // HD == c).astype(dtype)


def _expand_matrix(dtype=jnp.float32):
    r = lax.broadcasted_iota(jnp.int32, (NH, HID), 0)
    c = lax.broadcasted_iota(jnp.int32, (NH, HID), 1)
    return (c // HD == r).astype(dtype)


def _node_mlp_body(h_ref, w1t, b1, g1, be1, w2t, b2, o_ref):
    x = h_ref[...]
    h1 = jnp.dot(x, w1t[...], preferred_element_type=jnp.float32) + b1[...]
    h1 = jnp.maximum(_ln(h1, g1[...], be1[...]), 0.0)
    o_ref[...] = jnp.dot(h1, w2t[...], preferred_element_type=jnp.float32) + b2[...]


def _node_mlp(p, h):
    full = lambda s: pl.BlockSpec(s, lambda i: (0, 0))
    return pl.pallas_call(
        _node_mlp_body,
        grid=(NNB,),
        in_specs=[
            pl.BlockSpec((NB, HID), lambda i: (i, 0)),
            full((HID, HID)), full((1, HID)), full((1, HID)), full((1, HID)),
            full((HID, HID)), full((1, HID)),
        ],
        out_specs=pl.BlockSpec((NB, HID), lambda i: (i, 0)),
        out_shape=jax.ShapeDtypeStruct((N, HID), jnp.float32),
    )(h, p['W1'].T, p['b1'].reshape(1, -1), p['g1'].reshape(1, -1),
      p['be1'].reshape(1, -1), p['W2'].T, p['b2'].reshape(1, -1))


def _make_gather(widths, dtype=jnp.float32):
    nt = len(widths)

    def body(*refs):
        idx_hbm = refs[:nt]
        tab_hbm = refs[nt:2 * nt]
        out_hbm = refs[2 * nt:3 * nt]
        scratch = refs[3 * nt:]
        idx_v = [scratch[:nt], scratch[nt:2 * nt]]
        row_v = [scratch[2 * nt:3 * nt], scratch[3 * nt:4 * nt]]
        sem = [scratch[4 * nt], scratch[4 * nt + 1]]
        cid = lax.axis_index("c")
        sid = lax.axis_index("s")
        wid = sid * 2 + cid

        def load_and_start(i, b):
            base = wid * EPW + i * GC
            for t in range(nt):
                pltpu.sync_copy(idx_hbm[t].at[pl.ds(base, GC)],
                                idx_v[b][t].at[0])
            for t in range(nt):
                pltpu.make_async_copy(tab_hbm[t].at[idx_v[b][t].at[0]],
                                      row_v[b][t], sem[b]).start()

        def drain(i, b):
            base = wid * EPW + i * GC
            for t in range(nt):
                pltpu.make_async_copy(tab_hbm[t].at[idx_v[b][t].at[0]],
                                      row_v[b][t], sem[b]).wait()
            for t in range(nt):
                pltpu.sync_copy(row_v[b][t], out_hbm[t].at[pl.ds(base, GC)])

        load_and_start(0, 0)

        def chunk(i, carry):
            @pl.when(i + 1 < NCH)
            def _():
                @pl.when(lax.rem(i, 2) == 0)
                def _():
                    load_and_start(i + 1, 1)

                @pl.when(lax.rem(i, 2) == 1)
                def _():
                    load_and_start(i + 1, 0)

            @pl.when(lax.rem(i, 2) == 0)
            def _():
                drain(i, 0)

            @pl.when(lax.rem(i, 2) == 1)
            def _():
                drain(i, 1)

            return carry

        lax.fori_loop(0, NCH, chunk, 0)

    scratch = ([pltpu.VMEM((1, GC), jnp.int32) for _ in range(2 * nt)]
               + [pltpu.VMEM((GC, w), dtype) for w in widths]
               + [pltpu.VMEM((GC, w), dtype) for w in widths]
               + [pltpu.SemaphoreType.DMA, pltpu.SemaphoreType.DMA])
    out_type = [jax.ShapeDtypeStruct((E, w), dtype) for w in widths]

    built = {}

    def call(*args):
        if 'k' not in built:
            mesh = plsc.VectorSubcoreMesh(core_axis_name="c",
                                          subcore_axis_name="s")
            built['k'] = functools.partial(
                pl.kernel, mesh=mesh, out_type=out_type,
                scratch_types=scratch)(body)
        return built['k'](*args)

    return call


_gather2i = _make_gather([GW, GW], jnp.int32)


def _pack2bf(a, b):
    au = lax.bitcast_convert_type(a.astype(jnp.bfloat16), jnp.uint16)
    bu = lax.bitcast_convert_type(b.astype(jnp.bfloat16), jnp.uint16)
    w = (au.astype(jnp.uint32) << 16) | bu.astype(jnp.uint32)
    return lax.bitcast_convert_type(w, jnp.int32)


def _unpack_hi(u):
    return lax.bitcast_convert_type(u & jnp.int32(-65536), jnp.float32)


def _unpack_lo(u):
    return lax.bitcast_convert_type(u << 16, jnp.float32)


def _make_scatter(width):
    def body(pay_hbm, didx_hbm, zeros_hbm, out0, out1, idx_v, pay_v, acc):
        cid = lax.axis_index("c")
        sid = lax.axis_index("s")
        wid = sid * 2 + cid
        pltpu.sync_copy(zeros_hbm.at[pl.ds(sid * ZR, ZR)],
                        acc.at[pl.ds(sid * ZR, ZR)])
        plsc.subcore_barrier()

        def chunk(i, carry):
            base = wid * EPW + i * GC
            pltpu.sync_copy(didx_hbm.at[pl.ds(base, GC)], idx_v.at[0])
            pltpu.sync_copy(pay_hbm.at[pl.ds(base, GC)], pay_v)
            pltpu.sync_copy(pay_v, acc.at[idx_v.at[0]], add=True)
            return carry

        lax.fori_loop(0, NCH, chunk, 0)
        plsc.subcore_barrier()

        @pl.when(cid == 0)
        def _():
            pltpu.sync_copy(acc.at[pl.ds(sid * ZR, ZR)],
                            out0.at[pl.ds(sid * ZR, ZR)])

        @pl.when(cid == 1)
        def _():
            pltpu.sync_copy(acc.at[pl.ds(sid * ZR, ZR)],
                            out1.at[pl.ds(sid * ZR, ZR)])

    out_type = [jax.ShapeDtypeStruct((NP, width), jnp.float32)] * 2
    scratch = [pltpu.VMEM((1, GC), jnp.int32),
               pltpu.VMEM((GC, width), jnp.float32),
               pltpu.VMEM_SHARED((NP, width), jnp.float32)]

    built = {}

    def call(*args):
        if 'k' not in built:
            mesh = plsc.VectorSubcoreMesh(core_axis_name="c",
                                          subcore_axis_name="s")
            built['k'] = functools.partial(
                pl.kernel, mesh=mesh, out_type=out_type,
                scratch_types=scratch)(body)
        return built['k'](*args)

    return call


_scatter128 = _make_scatter(PAY)


def _edge_body_a(off_ref, ef_ref, ps_ref, pd_ref,
                 kw1ed, kw1i, kw1j, kb1, kg1, kbe1, kw2t, kb2,
                 vw1ed, vw1i, vw1j, vb1, vg1, vbe1, vw2t, vb2,
                 eww, ewb,
                 v_out, l_out, gm_out):
    ps = ps_ref[...]
    pd = pd_ref[...]
    dx = (lax.bitcast_convert_type(pd[:, HID:], jnp.float32)
          - lax.bitcast_convert_type(ps[:, HID:], jnp.float32))
    dist = jnp.sqrt(jnp.sum(dx * dx, axis=1, keepdims=True))
    off = off_ref[...]
    g = jnp.exp(_COEFF * (dist - off) ** 2)
    ef = ef_ref[...]
    df = jnp.concatenate([ef[:, i:i + 1] * g for i in range(EF)], axis=1)
    edf = jnp.concatenate([ef, df], axis=1)
    hs = _unpack_hi(ps[:, :HID])
    pd = pd[:, :HID]
    hd = _unpack_hi(pd)

    def mlp(w1ed, w1i, w1j, b1, g1, be1, w2t, b2):
        pre = (jnp.dot(edf, w1ed[...], preferred_element_type=jnp.float32)
               + jnp.dot(hd, w1i[...], preferred_element_type=jnp.float32)
               + jnp.dot(hs, w1j[...], preferred_element_type=jnp.float32)
               + b1[...])
        h1 = jnp.maximum(_ln(pre, g1[...], be1[...]), 0.0)
        return jnp.dot(h1, w2t[...], preferred_element_type=jnp.float32) + b2[...]

    k = mlp(kw1ed, kw1i, kw1j, kb1, kg1, kbe1, kw2t, kb2)
    v = mlp(vw1ed, vw1i, vw1j, vb1, vg1, vbe1, vw2t, vb2)
    e_w = jax.nn.sigmoid(jnp.sum(df * eww[...], axis=1, keepdims=True)
                         + ewb[...])
    v_out[...] = v * e_w
    qd = _unpack_lo(pd)
    logits = jnp.dot((qd * k) * _INV_SQRT_HD, _seg_matrix(),
                     preferred_element_type=jnp.float32)
    l_out[...] = logits
    bm = jnp.max(logits, axis=0, keepdims=True)

    @pl.when(pl.program_id(0) == 0)
    def _():
        gm_out[...] = jnp.full((8, NH), -1e30, jnp.float32)

    gm_out[...] = jnp.maximum(gm_out[...], bm)


def _edge_a(ef, ps, pd, pk, pv, eww, ewb):
    eb = lambda w: pl.BlockSpec((EB, w), lambda i: (i, 0))
    full = lambda s: pl.BlockSpec(s, lambda i: (0,) * len(s))

    def wspecs():
        return [full((84, HID)), full((HID, HID)), full((HID, HID)),
                full((1, HID)), full((1, HID)), full((1, HID)),
                full((HID, HID)), full((1, HID))]

    def wargs(p):
        w1t = p['W1'].T
        return [w1t[0:84], w1t[84:212], w1t[212:340],
                p['b1'].reshape(1, -1), p['g1'].reshape(1, -1),
                p['be1'].reshape(1, -1), p['W2'].T, p['b2'].reshape(1, -1)]

    return pl.pallas_call(
        _edge_body_a,
        grid=(NEB,),
        in_specs=[full((1, NG)), eb(EF), eb(GW), eb(GW)]
                 + wspecs() + wspecs() + [full((1, RF)), full((1, 1))],
        out_specs=[eb(HID), eb(NH), full((8, NH))],
        out_shape=[jax.ShapeDtypeStruct((E, HID), jnp.float32),
                   jax.ShapeDtypeStruct((E, NH), jnp.float32),
                   jax.ShapeDtypeStruct((8, NH), jnp.float32)],
    )(jnp.asarray(_OFFSET).reshape(1, NG), ef, ps, pd,
      *wargs(pk), *wargs(pv), eww.reshape(1, RF), ewb.reshape(1, 1))


def _payload_body_a(l_ref, v_ref, gm_ref, n_ref, d_ref):
    gm = jnp.max(gm_ref[...], axis=0, keepdims=True)
    ex = jnp.exp(l_ref[...] - gm)
    exb = jnp.dot(ex, _expand_matrix(), preferred_element_type=jnp.float32)
    n_ref[...] = v_ref[...] * exb
    d_ref[...] = exb


def _payload_a(logits, v_ew, gmax):
    return pl.pallas_call(
        _payload_body_a,
        grid=(NEB,),
        in_specs=[pl.BlockSpec((EB, NH), lambda i: (i, 0)),
                  pl.BlockSpec((EB, HID), lambda i: (i, 0)),
                  pl.BlockSpec((8, NH), lambda i: (0, 0))],
        out_specs=[pl.BlockSpec((EB, PAY), lambda i: (i, 0)),
                   pl.BlockSpec((EB, PAY), lambda i: (i, 0))],
        out_shape=[jax.ShapeDtypeStruct((E, PAY), jnp.float32),
                   jax.ShapeDtypeStruct((E, PAY), jnp.float32)],
    )(logits, v_ew, gmax)


def _combine_body_a(n0_ref, n1_ref, d0_ref, d1_ref, h_ref,
                    w1a, w1b, b1, g1, be1, w2t, b2, o_ref):
    num = n0_ref[...] + n1_ref[...]
    dnb = d0_ref[...] + d1_ref[...]
    attn = num / jnp.where(dnb == 0.0, 1.0, dnb)
    h = h_ref[...]
    pre = (jnp.dot(attn, w1a[...], preferred_element_type=jnp.float32)
           + jnp.dot(h, w1b[...], preferred_element_type=jnp.float32)
           + b1[...])
    h1 = jnp.maximum(_ln(pre, g1[...], be1[...]), 0.0)
    o_ref[...] = (jnp.dot(h1, w2t[...], preferred_element_type=jnp.float32)
                  + b2[...] + h)


def _combine_a(n0, n1, d0, d1, h, p):
    full = lambda s: pl.BlockSpec(s, lambda i: (0, 0))
    nb = lambda w: pl.BlockSpec((NB, w), lambda i: (i, 0))
    w1t = p['W1'].T
    return pl.pallas_call(
        _combine_body_a,
        grid=(NNB,),
        in_specs=[nb(PAY), nb(PAY), nb(PAY), nb(PAY), nb(HID),
                  full((HID, HID)), full((HID, HID)), full((1, HID)),
                  full((1, HID)), full((1, HID)), full((HID, HID)),
                  full((1, HID))],
        out_specs=pl.BlockSpec((NB, HID), lambda i: (i, 0)),
        out_shape=jax.ShapeDtypeStruct((N, HID), jnp.float32),
    )(n0, n1, d0, d1, h, w1t[:HID], w1t[HID:], p['b1'].reshape(1, -1),
      p['g1'].reshape(1, -1), p['be1'].reshape(1, -1), p['W2'].T,
      p['b2'].reshape(1, -1))


def _edge_body_b(off_ref, ef_ref, ps_ref, pd_ref,
                 kw1ed, kw1i, kw1j, kb1, kg1, kbe1, kw2t, kb2,
                 vw1ed, vw1i, vw1j, vb1, vg1, vbe1, vw2t, vb2,
                 eww, ewb,
                 w_out, l_out, gm_out):
    ps = ps_ref[...]
    pd = pd_ref[...]
    dx = (lax.bitcast_convert_type(pd[:, HID:], jnp.float32)
          - lax.bitcast_convert_type(ps[:, HID:], jnp.float32))
    dist = jnp.sqrt(jnp.sum(dx * dx, axis=1, keepdims=True))
    off = off_ref[...]
    g = jnp.exp(_COEFF * (dist - off) ** 2)
    ef = ef_ref[...]
    df = jnp.concatenate([ef[:, i:i + 1] * g for i in range(EF)], axis=1)
    edf = jnp.concatenate([ef, df], axis=1)
    hs = _unpack_hi(ps[:, :HID])
    pd = pd[:, :HID]
    hd = _unpack_hi(pd)

    def mlp(w1ed, w1i, w1j, b1, g1, be1, w2t, b2):
        pre = (jnp.dot(edf, w1ed[...], preferred_element_type=jnp.float32)
               + jnp.dot(hd, w1i[...], preferred_element_type=jnp.float32)
               + jnp.dot(hs, w1j[...], preferred_element_type=jnp.float32)
               + b1[...])
        h1 = jnp.maximum(_ln(pre, g1[...], be1[...]), 0.0)
        return jnp.dot(h1, w2t[...], preferred_element_type=jnp.float32) + b2[...]

    k = mlp(kw1ed, kw1i, kw1j, kb1, kg1, kbe1, kw2t, kb2)
    v2 = mlp(vw1ed, vw1i, vw1j, vb1, vg1, vbe1, vw2t, vb2)
    e_w = jax.nn.sigmoid(jnp.sum(df * eww[...], axis=1, keepdims=True)
                         + ewb[...])
    w_out[...] = v2 * e_w
    qd = _unpack_lo(pd)
    logits = jnp.dot((qd * k) * _INV_SQRT_HD, _seg_matrix(),
                     preferred_element_type=jnp.float32)
    l_out[...] = logits
    bm = jnp.max(logits, axis=0, keepdims=True)

    @pl.when(pl.program_id(0) == 0)
    def _():
        gm_out[...] = jnp.full((8, NH), -1e30, jnp.float32)

    gm_out[...] = jnp.maximum(gm_out[...], bm)


def _edge_b(ef, ps, pd, pk, pv, eww, ewb):
    eb = lambda w: pl.BlockSpec((EB, w), lambda i: (i, 0))
    full = lambda s: pl.BlockSpec(s, lambda i: (0,) * len(s))

    def wspecs(outw):
        return [full((84, HID)), full((HID, HID)), full((HID, HID)),
                full((1, HID)), full((1, HID)), full((1, HID)),
                full((HID, outw)), full((1, outw))]

    def wargs(p):
        w1t = p['W1'].T
        return [w1t[0:84], w1t[84:212], w1t[212:340],
                p['b1'].reshape(1, -1), p['g1'].reshape(1, -1),
                p['be1'].reshape(1, -1), p['W2'].T, p['b2'].reshape(1, -1)]

    return pl.pallas_call(
        _edge_body_b,
        grid=(NEB,),
        in_specs=[full((1, NG)), eb(EF), eb(GW), eb(GW)]
                 + wspecs(HID) + wspecs(NH) + [full((1, RF)), full((1, 1))],
        out_specs=[eb(NH), eb(NH), full((8, NH))],
        out_shape=[jax.ShapeDtypeStruct((E, NH), jnp.float32),
                   jax.ShapeDtypeStruct((E, NH), jnp.float32),
                   jax.ShapeDtypeStruct((8, NH), jnp.float32)],
    )(jnp.asarray(_OFFSET).reshape(1, NG), ef, ps, pd,
      *wargs(pk), *wargs(pv), eww.reshape(1, RF), ewb.reshape(1, 1))


def _payload_body_b(l_ref, w_ref, ps_ref, pd_ref, gm_ref, o_ref):
    gm = jnp.max(gm_ref[...], axis=0, keepdims=True)
    ex = jnp.exp(l_ref[...] - gm)
    p = ex * w_ref[...]
    rel = (lax.bitcast_convert_type(pd_ref[...], jnp.float32)
           - lax.bitcast_convert_type(ps_ref[...], jnp.float32))
    o_ref[...] = jnp.concatenate(
        [p * rel[:, 0:1], p * rel[:, 1:2], p * rel[:, 2:3], ex,
         jnp.zeros_like(rel[:, :64])], axis=1)


def _payload_b(logits, w2, ps, pd, gmax):
    return pl.pallas_call(
        _payload_body_b,
        grid=(NEB,),
        in_specs=[pl.BlockSpec((EB, NH), lambda i: (i, 0)),
                  pl.BlockSpec((EB, NH), lambda i: (i, 0)),
                  pl.BlockSpec((EB, XP), lambda i: (i, 1)),
                  pl.BlockSpec((EB, XP), lambda i: (i, 1)),
                  pl.BlockSpec((8, NH), lambda i: (0, 0))],
        out_specs=pl.BlockSpec((EB, PAY), lambda i: (i, 0)),
        out_shape=jax.ShapeDtypeStruct((E, PAY), jnp.float32),
    )(logits, w2, ps, pd, gmax)


def _final_body_b(a0_ref, a1_ref, xp_ref, m_ref, o_ref):
    a0 = a0_ref[...]
    a1 = a1_ref[...]
    dn = a0[:, 48:64] + a1[:, 48:64]
    dns = jnp.where(dn == 0.0, 1.0, dn)
    outs = []
    for c in range(3):
        nc = a0[:, c * NH:(c + 1) * NH] + a1[:, c * NH:(c + 1) * NH]
        outs.append(jnp.sum(nc / dns, axis=1, keepdims=True) * (1.0 / NH))
    delta = jnp.concatenate(outs + [jnp.zeros_like(a0[:, :125])], axis=1)
    o_ref[...] = xp_ref[...] + delta * m_ref[...]


def _final_b(acc0, acc1, x_pad, mask):
    return pl.pallas_call(
        _final_body_b,
        grid=(NNB,),
        in_specs=[pl.BlockSpec((NB, PAY), lambda i: (i, 0)),
                  pl.BlockSpec((NB, PAY), lambda i: (i, 0)),
                  pl.BlockSpec((NB, XP), lambda i: (i, 0)),
                  pl.BlockSpec((NB, 1), lambda i: (i, 0))],
        out_specs=pl.BlockSpec((NB, XP), lambda i: (i, 0)),
        out_shape=jax.ShapeDtypeStruct((N, XP), jnp.float32),
    )(acc0, acc1, x_pad, mask)


def kernel(h, x, edge_attr, edge_index, mask_ligand, params):
    src = jnp.asarray(edge_index[0], jnp.int32)
    dst = jnp.asarray(edge_index[1], jnp.int32)
    x_pad = jnp.pad(x, ((0, 0), (0, XP - 3)))
    x_i = lax.bitcast_convert_type(x_pad, jnp.int32)
    zeros = jnp.zeros((NP, PAY), jnp.float32)
    p = params

    q = _node_mlp(p['hq'], h)
    tk1 = jnp.concatenate([_pack2bf(h, q), x_i], axis=1)
    ps, pd = _gather2i(src, dst, tk1, tk1)
    v_ew, logits, gmax = _edge_a(edge_attr, ps, pd,
                                 p['hk'], p['hv'],
                                 p['ew_x2h_W'], p['ew_x2h_b'])
    num, den = _payload_a(logits, v_ew, gmax)
    n0, n1 = _scatter128(num, dst, zeros)
    d0, d1 = _scatter128(den, dst, zeros)
    new_h = _combine_a(n0[:N], n1[:N], d0[:N], d1[:N], h, p['node_out'])

    q2 = _node_mlp(p['xq'], new_h)
    tk2 = jnp.concatenate([_pack2bf(new_h, q2), x_i], axis=1)
    ps2, pd2 = _gather2i(src, dst, tk2, tk2)
    w2, logits2, gmax2 = _edge_b(edge_attr, ps2, pd2,
                                 p['xk'], p['xv'],
                                 p['ew_h2x_W'], p['ew_h2x_b'])
    pay2 = _payload_b(logits2, w2, ps2, pd2, gmax2)
    b0, b1 = _scatter128(pay2, dst, zeros)
    x_out = _final_b(b0[:N], b1[:N], x_pad, mask_ligand.reshape(-1, 1))

    return (new_h, x_out[:, :3])


# speedup vs baseline: 1.2723x; 1.0916x over previous
import functools

import jax
import jax.numpy as jnp
import numpy as np
from jax import lax
from jax.experimental import pallas as pl
from jax.experimental.pallas import tpu as pltpu
from jax.experimental.pallas import tpu_sc as plsc

N = 10000
E = 320000
HID = 128
NH = 16
HD = 8
NG = 20
EF = 4
RF = 80

_OFFSET = np.array([0, 1, 1.25, 1.5, 1.75, 2, 2.25, 2.5, 2.75, 3, 3.5, 4,
                    4.5, 5, 5.5, 6, 7, 8, 9, 10], dtype=np.float32)
_COEFF = -0.5
_INV_SQRT_HD = 1.0 / np.sqrt(HD)

EB = 5000
NEB = E // EB
NB = 2000
NNB = N // NB

NW = 32
EPW = E // NW
GC = 80
NCH = EPW // GC
NP = 10240
ZR = NP // 16

PAY = 128
XP = 128
GW = 256


def _ln(xv, g, b):
    mu = jnp.mean(xv, axis=-1, keepdims=True)
    c = xv - mu
    v = jnp.mean(c * c, axis=-1, keepdims=True)
    return c * lax.rsqrt(v + 1e-5) * g + b


def _seg_matrix(dtype=jnp.float32):
    r = lax.broadcasted_iota(jnp.int32, (HID, NH), 0)
    c = lax.broadcasted_iota(jnp.int32, (HID, NH), 1)
    return (r // HD == c).astype(dtype)


def _expand_matrix(dtype=jnp.float32):
    r = lax.broadcasted_iota(jnp.int32, (NH, HID), 0)
    c = lax.broadcasted_iota(jnp.int32, (NH, HID), 1)
    return (c // HD == r).astype(dtype)


def _node_mlp_body(h_ref, w1t, b1, g1, be1, w2t, b2, o_ref):
    x = h_ref[...]
    h1 = jnp.dot(x, w1t[...], preferred_element_type=jnp.float32) + b1[...]
    h1 = jnp.maximum(_ln(h1, g1[...], be1[...]), 0.0)
    o_ref[...] = jnp.dot(h1, w2t[...], preferred_element_type=jnp.float32) + b2[...]


def _node_mlp(p, h):
    full = lambda s: pl.BlockSpec(s, lambda i: (0, 0))
    return pl.pallas_call(
        _node_mlp_body,
        grid=(NNB,),
        in_specs=[
            pl.BlockSpec((NB, HID), lambda i: (i, 0)),
            full((HID, HID)), full((1, HID)), full((1, HID)), full((1, HID)),
            full((HID, HID)), full((1, HID)),
        ],
        out_specs=pl.BlockSpec((NB, HID), lambda i: (i, 0)),
        out_shape=jax.ShapeDtypeStruct((N, HID), jnp.float32),
    )(h, p['W1'].T, p['b1'].reshape(1, -1), p['g1'].reshape(1, -1),
      p['be1'].reshape(1, -1), p['W2'].T, p['b2'].reshape(1, -1))


def _make_gather(widths, dtype=jnp.float32):
    nt = len(widths)

    def body(*refs):
        idx_hbm = refs[:nt]
        tab_hbm = refs[nt:2 * nt]
        out_hbm = refs[2 * nt:3 * nt]
        scratch = refs[3 * nt:]
        idx_v = [scratch[:nt], scratch[nt:2 * nt]]
        row_v = [scratch[2 * nt:3 * nt], scratch[3 * nt:4 * nt]]
        sem = [scratch[4 * nt], scratch[4 * nt + 1]]
        cid = lax.axis_index("c")
        sid = lax.axis_index("s")
        wid = sid * 2 + cid

        def load_and_start(i, b):
            base = wid * EPW + i * GC
            for t in range(nt):
                pltpu.sync_copy(idx_hbm[t].at[pl.ds(base, GC)],
                                idx_v[b][t].at[0])
            for t in range(nt):
                pltpu.make_async_copy(tab_hbm[t].at[idx_v[b][t].at[0]],
                                      row_v[b][t], sem[b]).start()

        def drain(i, b):
            base = wid * EPW + i * GC
            for t in range(nt):
                pltpu.make_async_copy(tab_hbm[t].at[idx_v[b][t].at[0]],
                                      row_v[b][t], sem[b]).wait()
            for t in range(nt):
                pltpu.sync_copy(row_v[b][t], out_hbm[t].at[pl.ds(base, GC)])

        load_and_start(0, 0)

        def chunk(i, carry):
            @pl.when(i + 1 < NCH)
            def _():
                @pl.when(lax.rem(i, 2) == 0)
                def _():
                    load_and_start(i + 1, 1)

                @pl.when(lax.rem(i, 2) == 1)
                def _():
                    load_and_start(i + 1, 0)

            @pl.when(lax.rem(i, 2) == 0)
            def _():
                drain(i, 0)

            @pl.when(lax.rem(i, 2) == 1)
            def _():
                drain(i, 1)

            return carry

        lax.fori_loop(0, NCH, chunk, 0)

    scratch = ([pltpu.VMEM((1, GC), jnp.int32) for _ in range(2 * nt)]
               + [pltpu.VMEM((GC, w), dtype) for w in widths]
               + [pltpu.VMEM((GC, w), dtype) for w in widths]
               + [pltpu.SemaphoreType.DMA, pltpu.SemaphoreType.DMA])
    out_type = [jax.ShapeDtypeStruct((E, w), dtype) for w in widths]

    built = {}

    def call(*args):
        if 'k' not in built:
            mesh = plsc.VectorSubcoreMesh(core_axis_name="c",
                                          subcore_axis_name="s")
            built['k'] = functools.partial(
                pl.kernel, mesh=mesh, out_type=out_type,
                scratch_types=scratch)(body)
        return built['k'](*args)

    return call


_gather2i = _make_gather([GW, GW], jnp.int32)


def _pack2bf(a, b):
    au = lax.bitcast_convert_type(a.astype(jnp.bfloat16), jnp.uint16)
    bu = lax.bitcast_convert_type(b.astype(jnp.bfloat16), jnp.uint16)
    w = (au.astype(jnp.uint32) << 16) | bu.astype(jnp.uint32)
    return lax.bitcast_convert_type(w, jnp.int32)


def _unpack_hi(u):
    return lax.bitcast_convert_type(u & jnp.int32(-65536), jnp.float32)


def _unpack_lo(u):
    return lax.bitcast_convert_type(u << 16, jnp.float32)


def _make_scatter(width):
    def body(pay_hbm, didx_hbm, zeros_hbm, out0, out1,
             idx0, idx1, pay0, pay1, sem0, sem1, acc):
        idx_v = [idx0, idx1]
        pay_v = [pay0, pay1]
        sem = [sem0, sem1]
        cid = lax.axis_index("c")
        sid = lax.axis_index("s")
        wid = sid * 2 + cid
        pltpu.sync_copy(zeros_hbm.at[pl.ds(sid * ZR, ZR)],
                        acc.at[pl.ds(sid * ZR, ZR)])
        plsc.subcore_barrier()

        def load(i, b):
            base = wid * EPW + i * GC
            pltpu.sync_copy(didx_hbm.at[pl.ds(base, GC)], idx_v[b].at[0])
            pltpu.make_async_copy(pay_hbm.at[pl.ds(base, GC)],
                                  pay_v[b], sem[b]).start()

        def add(i, b):
            base = wid * EPW + i * GC
            pltpu.make_async_copy(pay_hbm.at[pl.ds(base, GC)],
                                  pay_v[b], sem[b]).wait()
            pltpu.sync_copy(pay_v[b], acc.at[idx_v[b].at[0]], add=True)

        load(0, 0)

        def chunk(i, carry):
            @pl.when(i + 1 < NCH)
            def _():
                @pl.when(lax.rem(i, 2) == 0)
                def _():
                    load(i + 1, 1)

                @pl.when(lax.rem(i, 2) == 1)
                def _():
                    load(i + 1, 0)

            @pl.when(lax.rem(i, 2) == 0)
            def _():
                add(i, 0)

            @pl.when(lax.rem(i, 2) == 1)
            def _():
                add(i, 1)

            return carry

        lax.fori_loop(0, NCH, chunk, 0)
        plsc.subcore_barrier()

        @pl.when(cid == 0)
        def _():
            pltpu.sync_copy(acc.at[pl.ds(sid * ZR, ZR)],
                            out0.at[pl.ds(sid * ZR, ZR)])

        @pl.when(cid == 1)
        def _():
            pltpu.sync_copy(acc.at[pl.ds(sid * ZR, ZR)],
                            out1.at[pl.ds(sid * ZR, ZR)])

    out_type = [jax.ShapeDtypeStruct((NP, width), jnp.float32)] * 2
    scratch = [pltpu.VMEM((1, GC), jnp.int32),
               pltpu.VMEM((1, GC), jnp.int32),
               pltpu.VMEM((GC, width), jnp.float32),
               pltpu.VMEM((GC, width), jnp.float32),
               pltpu.SemaphoreType.DMA, pltpu.SemaphoreType.DMA,
               pltpu.VMEM_SHARED((NP, width), jnp.float32)]

    built = {}

    def call(*args):
        if 'k' not in built:
            mesh = plsc.VectorSubcoreMesh(core_axis_name="c",
                                          subcore_axis_name="s")
            built['k'] = functools.partial(
                pl.kernel, mesh=mesh, out_type=out_type,
                scratch_types=scratch)(body)
        return built['k'](*args)

    return call


_scatter128 = _make_scatter(PAY)


def _edge_body_a(off_ref, ef_ref, ps_ref, pd_ref,
                 kw1ed, kw1i, kw1j, kb1, kg1, kbe1, kw2t, kb2,
                 vw1ed, vw1i, vw1j, vb1, vg1, vbe1, vw2t, vb2,
                 eww, ewb,
                 v_out, l_out, gm_out):
    ps = ps_ref[...]
    pd = pd_ref[...]
    dx = (lax.bitcast_convert_type(pd[:, HID:], jnp.float32)
          - lax.bitcast_convert_type(ps[:, HID:], jnp.float32))
    dist = jnp.sqrt(jnp.sum(dx * dx, axis=1, keepdims=True))
    off = off_ref[...]
    g = jnp.exp(_COEFF * (dist - off) ** 2)
    ef = ef_ref[...]
    df = jnp.concatenate([ef[:, i:i + 1] * g for i in range(EF)], axis=1)
    edf = jnp.concatenate([ef, df], axis=1)
    hs = _unpack_hi(ps[:, :HID])
    pd = pd[:, :HID]
    hd = _unpack_hi(pd)

    def mlp(w1ed, w1i, w1j, b1, g1, be1, w2t, b2):
        pre = (jnp.dot(edf, w1ed[...], preferred_element_type=jnp.float32)
               + jnp.dot(hd, w1i[...], preferred_element_type=jnp.float32)
               + jnp.dot(hs, w1j[...], preferred_element_type=jnp.float32)
               + b1[...])
        h1 = jnp.maximum(_ln(pre, g1[...], be1[...]), 0.0)
        return jnp.dot(h1, w2t[...], preferred_element_type=jnp.float32) + b2[...]

    k = mlp(kw1ed, kw1i, kw1j, kb1, kg1, kbe1, kw2t, kb2)
    v = mlp(vw1ed, vw1i, vw1j, vb1, vg1, vbe1, vw2t, vb2)
    e_w = jax.nn.sigmoid(jnp.sum(df * eww[...], axis=1, keepdims=True)
                         + ewb[...])
    v_out[...] = v * e_w
    qd = _unpack_lo(pd)
    logits = jnp.dot((qd * k) * _INV_SQRT_HD, _seg_matrix(),
                     preferred_element_type=jnp.float32)
    l_out[...] = logits
    bm = jnp.max(logits, axis=0, keepdims=True)

    @pl.when(pl.program_id(0) == 0)
    def _():
        gm_out[...] = jnp.full((8, NH), -1e30, jnp.float32)

    gm_out[...] = jnp.maximum(gm_out[...], bm)


def _edge_a(ef, ps, pd, pk, pv, eww, ewb):
    eb = lambda w: pl.BlockSpec((EB, w), lambda i: (i, 0))
    full = lambda s: pl.BlockSpec(s, lambda i: (0,) * len(s))

    def wspecs():
        return [full((84, HID)), full((HID, HID)), full((HID, HID)),
                full((1, HID)), full((1, HID)), full((1, HID)),
                full((HID, HID)), full((1, HID))]

    def wargs(p):
        w1t = p['W1'].T
        return [w1t[0:84], w1t[84:212], w1t[212:340],
                p['b1'].reshape(1, -1), p['g1'].reshape(1, -1),
                p['be1'].reshape(1, -1), p['W2'].T, p['b2'].reshape(1, -1)]

    return pl.pallas_call(
        _edge_body_a,
        grid=(NEB,),
        in_specs=[full((1, NG)), eb(EF), eb(GW), eb(GW)]
                 + wspecs() + wspecs() + [full((1, RF)), full((1, 1))],
        out_specs=[eb(HID), eb(NH), full((8, NH))],
        out_shape=[jax.ShapeDtypeStruct((E, HID), jnp.float32),
                   jax.ShapeDtypeStruct((E, NH), jnp.float32),
                   jax.ShapeDtypeStruct((8, NH), jnp.float32)],
    )(jnp.asarray(_OFFSET).reshape(1, NG), ef, ps, pd,
      *wargs(pk), *wargs(pv), eww.reshape(1, RF), ewb.reshape(1, 1))


def _payload_body_a(l_ref, v_ref, gm_ref, n_ref, d_ref):
    gm = jnp.max(gm_ref[...], axis=0, keepdims=True)
    ex = jnp.exp(l_ref[...] - gm)
    exb = jnp.dot(ex, _expand_matrix(), preferred_element_type=jnp.float32)
    n_ref[...] = v_ref[...] * exb
    d_ref[...] = exb


def _payload_a(logits, v_ew, gmax):
    return pl.pallas_call(
        _payload_body_a,
        grid=(NEB,),
        in_specs=[pl.BlockSpec((EB, NH), lambda i: (i, 0)),
                  pl.BlockSpec((EB, HID), lambda i: (i, 0)),
                  pl.BlockSpec((8, NH), lambda i: (0, 0))],
        out_specs=[pl.BlockSpec((EB, PAY), lambda i: (i, 0)),
                   pl.BlockSpec((EB, PAY), lambda i: (i, 0))],
        out_shape=[jax.ShapeDtypeStruct((E, PAY), jnp.float32),
                   jax.ShapeDtypeStruct((E, PAY), jnp.float32)],
    )(logits, v_ew, gmax)


def _combine_body_a(n0_ref, n1_ref, d0_ref, d1_ref, h_ref,
                    w1a, w1b, b1, g1, be1, w2t, b2, o_ref):
    num = n0_ref[...] + n1_ref[...]
    dnb = d0_ref[...] + d1_ref[...]
    attn = num / jnp.where(dnb == 0.0, 1.0, dnb)
    h = h_ref[...]
    pre = (jnp.dot(attn, w1a[...], preferred_element_type=jnp.float32)
           + jnp.dot(h, w1b[...], preferred_element_type=jnp.float32)
           + b1[...])
    h1 = jnp.maximum(_ln(pre, g1[...], be1[...]), 0.0)
    o_ref[...] = (jnp.dot(h1, w2t[...], preferred_element_type=jnp.float32)
                  + b2[...] + h)


def _combine_a(n0, n1, d0, d1, h, p):
    full = lambda s: pl.BlockSpec(s, lambda i: (0, 0))
    nb = lambda w: pl.BlockSpec((NB, w), lambda i: (i, 0))
    w1t = p['W1'].T
    return pl.pallas_call(
        _combine_body_a,
        grid=(NNB,),
        in_specs=[nb(PAY), nb(PAY), nb(PAY), nb(PAY), nb(HID),
                  full((HID, HID)), full((HID, HID)), full((1, HID)),
                  full((1, HID)), full((1, HID)), full((HID, HID)),
                  full((1, HID))],
        out_specs=pl.BlockSpec((NB, HID), lambda i: (i, 0)),
        out_shape=jax.ShapeDtypeStruct((N, HID), jnp.float32),
    )(n0, n1, d0, d1, h, w1t[:HID], w1t[HID:], p['b1'].reshape(1, -1),
      p['g1'].reshape(1, -1), p['be1'].reshape(1, -1), p['W2'].T,
      p['b2'].reshape(1, -1))


def _edge_body_b(off_ref, ef_ref, ps_ref, pd_ref,
                 kw1ed, kw1i, kw1j, kb1, kg1, kbe1, kw2t, kb2,
                 vw1ed, vw1i, vw1j, vb1, vg1, vbe1, vw2t, vb2,
                 eww, ewb,
                 w_out, l_out, gm_out):
    ps = ps_ref[...]
    pd = pd_ref[...]
    dx = (lax.bitcast_convert_type(pd[:, HID:], jnp.float32)
          - lax.bitcast_convert_type(ps[:, HID:], jnp.float32))
    dist = jnp.sqrt(jnp.sum(dx * dx, axis=1, keepdims=True))
    off = off_ref[...]
    g = jnp.exp(_COEFF * (dist - off) ** 2)
    ef = ef_ref[...]
    df = jnp.concatenate([ef[:, i:i + 1] * g for i in range(EF)], axis=1)
    edf = jnp.concatenate([ef, df], axis=1)
    hs = _unpack_hi(ps[:, :HID])
    pd = pd[:, :HID]
    hd = _unpack_hi(pd)

    def mlp(w1ed, w1i, w1j, b1, g1, be1, w2t, b2):
        pre = (jnp.dot(edf, w1ed[...], preferred_element_type=jnp.float32)
               + jnp.dot(hd, w1i[...], preferred_element_type=jnp.float32)
               + jnp.dot(hs, w1j[...], preferred_element_type=jnp.float32)
               + b1[...])
        h1 = jnp.maximum(_ln(pre, g1[...], be1[...]), 0.0)
        return jnp.dot(h1, w2t[...], preferred_element_type=jnp.float32) + b2[...]

    k = mlp(kw1ed, kw1i, kw1j, kb1, kg1, kbe1, kw2t, kb2)
    v2 = mlp(vw1ed, vw1i, vw1j, vb1, vg1, vbe1, vw2t, vb2)
    e_w = jax.nn.sigmoid(jnp.sum(df * eww[...], axis=1, keepdims=True)
                         + ewb[...])
    w_out[...] = v2 * e_w
    qd = _unpack_lo(pd)
    logits = jnp.dot((qd * k) * _INV_SQRT_HD, _seg_matrix(),
                     preferred_element_type=jnp.float32)
    l_out[...] = logits
    bm = jnp.max(logits, axis=0, keepdims=True)

    @pl.when(pl.program_id(0) == 0)
    def _():
        gm_out[...] = jnp.full((8, NH), -1e30, jnp.float32)

    gm_out[...] = jnp.maximum(gm_out[...], bm)


def _edge_b(ef, ps, pd, pk, pv, eww, ewb):
    eb = lambda w: pl.BlockSpec((EB, w), lambda i: (i, 0))
    full = lambda s: pl.BlockSpec(s, lambda i: (0,) * len(s))

    def wspecs(outw):
        return [full((84, HID)), full((HID, HID)), full((HID, HID)),
                full((1, HID)), full((1, HID)), full((1, HID)),
                full((HID, outw)), full((1, outw))]

    def wargs(p):
        w1t = p['W1'].T
        return [w1t[0:84], w1t[84:212], w1t[212:340],
                p['b1'].reshape(1, -1), p['g1'].reshape(1, -1),
                p['be1'].reshape(1, -1), p['W2'].T, p['b2'].reshape(1, -1)]

    return pl.pallas_call(
        _edge_body_b,
        grid=(NEB,),
        in_specs=[full((1, NG)), eb(EF), eb(GW), eb(GW)]
                 + wspecs(HID) + wspecs(NH) + [full((1, RF)), full((1, 1))],
        out_specs=[eb(NH), eb(NH), full((8, NH))],
        out_shape=[jax.ShapeDtypeStruct((E, NH), jnp.float32),
                   jax.ShapeDtypeStruct((E, NH), jnp.float32),
                   jax.ShapeDtypeStruct((8, NH), jnp.float32)],
    )(jnp.asarray(_OFFSET).reshape(1, NG), ef, ps, pd,
      *wargs(pk), *wargs(pv), eww.reshape(1, RF), ewb.reshape(1, 1))


def _payload_body_b(l_ref, w_ref, ps_ref, pd_ref, gm_ref, o_ref):
    gm = jnp.max(gm_ref[...], axis=0, keepdims=True)
    ex = jnp.exp(l_ref[...] - gm)
    p = ex * w_ref[...]
    rel = (lax.bitcast_convert_type(pd_ref[...], jnp.float32)
           - lax.bitcast_convert_type(ps_ref[...], jnp.float32))
    o_ref[...] = jnp.concatenate(
        [p * rel[:, 0:1], p * rel[:, 1:2], p * rel[:, 2:3], ex,
         jnp.zeros_like(rel[:, :64])], axis=1)


def _payload_b(logits, w2, ps, pd, gmax):
    return pl.pallas_call(
        _payload_body_b,
        grid=(NEB,),
        in_specs=[pl.BlockSpec((EB, NH), lambda i: (i, 0)),
                  pl.BlockSpec((EB, NH), lambda i: (i, 0)),
                  pl.BlockSpec((EB, XP), lambda i: (i, 1)),
                  pl.BlockSpec((EB, XP), lambda i: (i, 1)),
                  pl.BlockSpec((8, NH), lambda i: (0, 0))],
        out_specs=pl.BlockSpec((EB, PAY), lambda i: (i, 0)),
        out_shape=jax.ShapeDtypeStruct((E, PAY), jnp.float32),
    )(logits, w2, ps, pd, gmax)


def _final_body_b(a0_ref, a1_ref, xp_ref, m_ref, o_ref):
    a0 = a0_ref[...]
    a1 = a1_ref[...]
    dn = a0[:, 48:64] + a1[:, 48:64]
    dns = jnp.where(dn == 0.0, 1.0, dn)
    outs = []
    for c in range(3):
        nc = a0[:, c * NH:(c + 1) * NH] + a1[:, c * NH:(c + 1) * NH]
        outs.append(jnp.sum(nc / dns, axis=1, keepdims=True) * (1.0 / NH))
    delta = jnp.concatenate(outs + [jnp.zeros_like(a0[:, :125])], axis=1)
    o_ref[...] = xp_ref[...] + delta * m_ref[...]


def _final_b(acc0, acc1, x_pad, mask):
    return pl.pallas_call(
        _final_body_b,
        grid=(NNB,),
        in_specs=[pl.BlockSpec((NB, PAY), lambda i: (i, 0)),
                  pl.BlockSpec((NB, PAY), lambda i: (i, 0)),
                  pl.BlockSpec((NB, XP), lambda i: (i, 0)),
                  pl.BlockSpec((NB, 1), lambda i: (i, 0))],
        out_specs=pl.BlockSpec((NB, XP), lambda i: (i, 0)),
        out_shape=jax.ShapeDtypeStruct((N, XP), jnp.float32),
    )(acc0, acc1, x_pad, mask)


def kernel(h, x, edge_attr, edge_index, mask_ligand, params):
    src = jnp.asarray(edge_index[0], jnp.int32)
    dst = jnp.asarray(edge_index[1], jnp.int32)
    x_pad = jnp.pad(x, ((0, 0), (0, XP - 3)))
    x_i = lax.bitcast_convert_type(x_pad, jnp.int32)
    zeros = jnp.zeros((NP, PAY), jnp.float32)
    p = params

    q = _node_mlp(p['hq'], h)
    tk1 = jnp.concatenate([_pack2bf(h, q), x_i], axis=1)
    ps, pd = _gather2i(src, dst, tk1, tk1)
    v_ew, logits, gmax = _edge_a(edge_attr, ps, pd,
                                 p['hk'], p['hv'],
                                 p['ew_x2h_W'], p['ew_x2h_b'])
    num, den = _payload_a(logits, v_ew, gmax)
    n0, n1 = _scatter128(num, dst, zeros)
    d0, d1 = _scatter128(den, dst, zeros)
    new_h = _combine_a(n0[:N], n1[:N], d0[:N], d1[:N], h, p['node_out'])

    q2 = _node_mlp(p['xq'], new_h)
    tk2 = jnp.concatenate([_pack2bf(new_h, q2), x_i], axis=1)
    ps2, pd2 = _gather2i(src, dst, tk2, tk2)
    w2, logits2, gmax2 = _edge_b(edge_attr, ps2, pd2,
                                 p['xk'], p['xv'],
                                 p['ew_h2x_W'], p['ew_h2x_b'])
    pay2 = _payload_b(logits2, w2, ps2, pd2, gmax2)
    b0, b1 = _scatter128(pay2, dst, zeros)
    x_out = _final_b(b0[:N], b1[:N], x_pad, mask_ligand.reshape(-1, 1))

    return (new_h, x_out[:, :3])
